# Initial kernel scaffold; baseline (speedup 1.0000x reference)
import jax
import jax.numpy as jnp
from jax.experimental import pallas as pl


def kernel(x, edge_index, batch, W1, att_src1, att_dst1, b1, W2, att_src2, att_dst2, b2, Wlin, blin):
    raise NotImplementedError("write your pallas kernel here")


# speedup vs baseline: 13.3761x; 13.3761x over previous
import functools

import jax
import jax.numpy as jnp
from jax import lax
from jax.experimental import pallas as pl
from jax.experimental.pallas import tpu as pltpu
from jax.experimental.pallas import tpu_sc as plsc

HEADS = 8
HID = 64
NG = 16
LANES = 16
NW = 32
KE = 128


def _tc1_body(x_ref, w_ref, as_ref, ad_ref, h_ref, asrc_ref, adst_ref):
    c = pl.program_id(1)
    h = jnp.dot(x_ref[...], w_ref[...], preferred_element_type=jnp.float32)
    h_ref[0] = h

    @pl.when(c == 0)
    def _():
        asrc_ref[...] = jnp.zeros_like(asrc_ref)
        adst_ref[...] = jnp.zeros_like(adst_ref)

    asrc_ref[...] += jnp.dot(h, as_ref[...], preferred_element_type=jnp.float32)
    adst_ref[...] += jnp.dot(h, ad_ref[...], preferred_element_type=jnp.float32)


def _tc2_body(outc_ref, den_ref, b1_ref, w2_ref, as2_ref, ad2_ref, e2_ref,
              h2a_ref, adst2_ref):
    chunks = []
    for c in range(4):
        raw = outc_ref[c]
        d = den_ref[0, :, 2 * c:2 * c + 2] + den_ref[1, :, 2 * c:2 * c + 2]
        db = jnp.dot(d + 1e-16, e2_ref[...],
                     preferred_element_type=jnp.float32)
        z = raw / db + b1_ref[0, c * 128:(c + 1) * 128]
        chunks.append(jnp.where(z > 0, z, jnp.exp(z) - 1.0))
    h1 = jnp.concatenate(chunks, axis=1)
    h2 = jnp.dot(h1, w2_ref[...], preferred_element_type=jnp.float32)
    a2s = jnp.dot(h2, as2_ref[...], preferred_element_type=jnp.float32)
    a2d = jnp.dot(h2, ad2_ref[...], preferred_element_type=jnp.float32)
    bn = h2.shape[0]
    h2a_ref[...] = jnp.concatenate(
        [h2, jnp.broadcast_to(a2s, (bn, LANES))], axis=1)
    adst2_ref[...] = jnp.broadcast_to(a2d, (bn, LANES))


def _tc3_body(acc_ref, batch_ref, b2_ref, wl_ref, bl_ref,
              out_ref, sums_ref, cnt_ref):
    i = pl.program_id(0)
    rows = acc_ref[0] + acc_ref[1]
    den = rows[:, HID:HID + 1] + 1e-16
    h2o = rows[:, :HID] / den + b2_ref[...]
    bn = h2o.shape[0]
    oh = (batch_ref[...] == lax.broadcasted_iota(jnp.int32, (1, NG), 1)
          ).astype(jnp.float32)

    @pl.when(i == 0)
    def _():
        sums_ref[...] = jnp.zeros_like(sums_ref)
        cnt_ref[...] = jnp.zeros_like(cnt_ref)

    dn = (((0,), (0,)), ((), ()))
    sums_ref[...] += lax.dot_general(oh, h2o, dn,
                                     preferred_element_type=jnp.float32)
    cnt_ref[...] += lax.dot_general(oh, jnp.ones((bn, HID), jnp.float32), dn,
                                    preferred_element_type=jnp.float32)
    pooled = sums_ref[...] / jnp.maximum(cnt_ref[...], 1.0)
    out_ref[...] = (jnp.dot(pooled, wl_ref[...],
                            preferred_element_type=jnp.float32) + bl_ref[...])


def _pipeline(x, edge_index, batch, W1, att_src1, att_dst1, b1,
              W2, att_src2, att_dst2, b2, Wlin, blin):
    f32, i32 = jnp.float32, jnp.int32
    N, F = x.shape
    E0 = edge_index.shape[1]
    ei = edge_index.astype(i32)

    NPAD = ((N + 255) // 256) * 256
    RPT = NPAD // 16
    E1 = E0 + N
    EPT = ((E1 + NW * KE - 1) // (NW * KE)) * KE
    NBLK = EPT // KE
    EPAD = EPT * NW

    loops = jnp.arange(N, dtype=i32)
    src = jnp.concatenate([ei[0], loops,
                           jnp.zeros((EPAD - E1,), i32)])
    dst = jnp.concatenate([ei[1], loops,
                           jnp.full((EPAD - E1,), N, i32)])
    xp = jnp.pad(x, ((0, NPAD - N), (0, 0)))
    batch_p = jnp.pad(batch.astype(i32), (0, NPAD - N),
                      constant_values=NG).reshape(NPAD, 1)

    eyeH = jnp.eye(HEADS, dtype=f32)
    A2s = (att_src1[0][:, :, None] * eyeH[:, None, :]).reshape(HEADS * HID, HEADS)
    A2d = (att_dst1[0][:, :, None] * eyeH[:, None, :]).reshape(HEADS * HID, HEADS)
    A2s = jnp.concatenate([A2s, A2s], axis=1)
    A2d = jnp.concatenate([A2d, A2d], axis=1)
    E2 = jnp.repeat(jnp.eye(2, dtype=f32), 128 // 2, axis=1)
    as2 = att_src2[0, 0].reshape(HID, 1)
    ad2 = att_dst2[0, 0].reshape(HID, 1)
    b1r = b1.reshape(1, HEADS * HID)
    b2r = b2.reshape(1, HID)
    blr = blin.reshape(1, -1)
    z16 = jnp.zeros((RPT, LANES), f32)
    z128 = jnp.zeros((RPT, 128), f32)
    z80 = jnp.zeros((RPT, 80), f32)

    BN = 256
    NB = NPAD // BN

    hc, asrc16, adst16 = pl.pallas_call(
        _tc1_body,
        grid=(NB, 4),
        in_specs=[
            pl.BlockSpec((BN, F), lambda i, c: (i, 0)),
            pl.BlockSpec((F, 128), lambda i, c: (0, c)),
            pl.BlockSpec((128, LANES), lambda i, c: (c, 0)),
            pl.BlockSpec((128, LANES), lambda i, c: (c, 0)),
        ],
        out_specs=[
            pl.BlockSpec((1, BN, 128), lambda i, c: (c, i, 0)),
            pl.BlockSpec((BN, LANES), lambda i, c: (i, 0)),
            pl.BlockSpec((BN, LANES), lambda i, c: (i, 0)),
        ],
        out_shape=[
            jax.ShapeDtypeStruct((4, NPAD, 128), f32),
            jax.ShapeDtypeStruct((NPAD, LANES), f32),
            jax.ShapeDtypeStruct((NPAD, LANES), f32),
        ],
    )(xp, W1, A2s, A2d)

    mesh = plsc.VectorSubcoreMesh(core_axis_name="c", subcore_axis_name="s")

    @functools.partial(
        pl.kernel,
        out_type=(jax.ShapeDtypeStruct((EPAD, LANES), f32),
                  jax.ShapeDtypeStruct((2 * NPAD, LANES), f32)),
        mesh=mesh,
        compiler_params=pltpu.CompilerParams(use_tc_tiling_on_sc=False),
        scratch_types=[
            pltpu.VMEM((KE,), i32), pltpu.VMEM((KE,), i32),
            pltpu.VMEM((KE, LANES), f32), pltpu.VMEM((KE, LANES), f32),
            pltpu.VMEM((KE, LANES), f32),
            pltpu.VMEM_SHARED((NPAD, LANES), f32),
            pltpu.SemaphoreType.DMA, pltpu.SemaphoreType.DMA,
        ],
    )
    def sc_a1(asrc_h, adst_h, src_h, dst_h, z16_h, ex_h, den_h,
              sidx, didx, asb, adb, exb, dacc, sem1, sem2):
        cid = lax.axis_index("c")
        sid = lax.axis_index("s")
        wid = cid * 16 + sid
        pltpu.sync_copy(z16_h, dacc.at[pl.ds(sid * RPT, RPT)])
        plsc.subcore_barrier()

        def blk(b, _):
            base = wid * EPT + b * KE
            pltpu.sync_copy(src_h.at[pl.ds(base, KE)], sidx)
            pltpu.sync_copy(dst_h.at[pl.ds(base, KE)], didx)
            cp1 = pltpu.async_copy(asrc_h.at[sidx], asb, sem1)
            cp2 = pltpu.async_copy(adst_h.at[didx], adb, sem2)
            cp1.wait()
            cp2.wait()

            def edge(e, _):
                s = asb[e, pl.ds(0, LANES)] + adb[e, pl.ds(0, LANES)]
                ev = jnp.exp(jnp.maximum(s, 0.2 * s))
                exb[e, pl.ds(0, LANES)] = ev
                return 0

            lax.fori_loop(0, KE, edge, 0)
            pltpu.sync_copy(exb, ex_h.at[pl.ds(base, KE)])
            pltpu.sync_copy(exb, dacc.at[didx], add=True)
            return 0

        lax.fori_loop(0, NBLK, blk, 0)
        plsc.subcore_barrier()
        pltpu.sync_copy(dacc.at[pl.ds(sid * RPT, RPT)],
                        den_h.at[pl.ds(cid * NPAD + sid * RPT, RPT)])

    ex, den2 = sc_a1(asrc16, adst16, src, dst, z16)

    EPT16 = EPAD // 16
    NBLK16 = EPT16 // KE
    @functools.partial(
        pl.kernel,
        out_type=jax.ShapeDtypeStruct((4 * NPAD * 8, LANES), f32),
        mesh=mesh,
        compiler_params=pltpu.CompilerParams(use_tc_tiling_on_sc=False),
        scratch_types=[
            pltpu.VMEM((KE,), i32), pltpu.VMEM((KE,), i32),
            pltpu.VMEM((KE,), i32),
            pltpu.VMEM((KE, 128), f32),
            [pltpu.VMEM((KE, LANES), f32)] * 8,
            pltpu.VMEM((KE, LANES), f32),
            pltpu.VMEM_SHARED((NPAD * 8, LANES), f32),
            pltpu.SemaphoreType.DMA,
        ],
    )
    def sc_b1(hcat_h, exf_h, src4_h, dst_h, z128_h, outc_h,
              sidx, didx, d8, hbuf, sbufs, exb, oacc, sem):
        cid = lax.axis_index("c")
        sid = lax.axis_index("s")
        for chunk in range(2):
            c = cid * 2 + chunk
            cN = c * NPAD
            c2 = c * 2
            pltpu.sync_copy(z128_h, oacc.at[pl.ds(sid * RPT * 8, RPT * 8)])
            plsc.subcore_barrier()

            def blk(b, _):
                base = sid * EPT16 + b * KE
                pltpu.sync_copy(src4_h.at[pl.ds(c * EPAD + base, KE)], sidx)
                pltpu.sync_copy(dst_h.at[pl.ds(base, KE)], didx)
                pltpu.sync_copy(exf_h.at[pl.ds(base, KE)], exb)
                pltpu.async_copy(hcat_h.at[sidx], hbuf, sem).wait()

                def edge(e, _):
                    row = exb[e, pl.ds(0, LANES)]
                    s0 = row.at[jnp.full((LANES,), c2, i32)].get(
                        mode="promise_in_bounds")
                    s1 = row.at[jnp.full((LANES,), c2 + 1, i32)].get(
                        mode="promise_in_bounds")
                    for r in range(8):
                        sv = s0 if r < 4 else s1
                        sl = pl.ds(r * LANES, LANES)
                        sbufs[r][e, pl.ds(0, LANES)] = hbuf[e, sl] * sv
                    return 0

                lax.fori_loop(0, KE, edge, 0)
                for r in range(8):
                    for j in range(KE // LANES):
                        sl = pl.ds(j * LANES, LANES)
                        d8[sl] = didx[sl] * 8 + r
                    pltpu.sync_copy(sbufs[r], oacc.at[d8], add=True)
                return 0

            lax.fori_loop(0, NBLK16, blk, 0)
            plsc.subcore_barrier()
            pltpu.sync_copy(oacc.at[pl.ds(sid * RPT * 8, RPT * 8)],
                            outc_h.at[pl.ds((cN + sid * RPT) * 8, RPT * 8)])

    src4 = jnp.concatenate([src + c * NPAD for c in range(4)])
    outc = sc_b1(hc.reshape(4 * NPAD, 128), ex, src4, dst,
                 z128.reshape(RPT * 8, LANES))

    h2a, adst2t = pl.pallas_call(
        _tc2_body,
        grid=(NB,),
        in_specs=[
            pl.BlockSpec((4, BN, 128), lambda i: (0, i, 0)),
            pl.BlockSpec((2, BN, LANES), lambda i: (0, i, 0)),
            pl.BlockSpec((1, HEADS * HID), lambda i: (0, 0)),
            pl.BlockSpec((HEADS * HID, HID), lambda i: (0, 0)),
            pl.BlockSpec((HID, 1), lambda i: (0, 0)),
            pl.BlockSpec((HID, 1), lambda i: (0, 0)),
            pl.BlockSpec((2, 128), lambda i: (0, 0)),
        ],
        out_specs=[
            pl.BlockSpec((BN, HID + LANES), lambda i: (i, 0)),
            pl.BlockSpec((BN, LANES), lambda i: (i, 0)),
        ],
        out_shape=[
            jax.ShapeDtypeStruct((NPAD, HID + LANES), f32),
            jax.ShapeDtypeStruct((NPAD, LANES), f32),
        ],
    )(outc.reshape(4, NPAD, 128), den2.reshape(2, NPAD, LANES), b1r, W2,
      as2, ad2, E2)

    @functools.partial(
        pl.kernel,
        out_type=jax.ShapeDtypeStruct((2 * NPAD * 5, LANES), f32),
        mesh=mesh,
        compiler_params=pltpu.CompilerParams(use_tc_tiling_on_sc=False),
        scratch_types=[
            pltpu.VMEM((KE,), i32), pltpu.VMEM((KE,), i32),
            pltpu.VMEM((KE,), i32),
            pltpu.VMEM((KE, 80), f32), pltpu.VMEM((KE, LANES), f32),
            [pltpu.VMEM((KE, LANES), f32)] * 5,
            pltpu.VMEM_SHARED((NPAD * 5, LANES), f32),
            pltpu.SemaphoreType.DMA, pltpu.SemaphoreType.DMA,
        ],
    )
    def sc_b2(h2a_h, adst_h, src_h, dst_h, z80_h, acc_h,
              sidx, didx, d5, hbuf, abuf, obufs, oacc, sem1, sem2):
        cid = lax.axis_index("c")
        sid = lax.axis_index("s")
        wid = cid * 16 + sid
        pltpu.sync_copy(z80_h, oacc.at[pl.ds(sid * RPT * 5, RPT * 5)])
        plsc.subcore_barrier()

        def blk(b, _):
            base = wid * EPT + b * KE
            pltpu.sync_copy(src_h.at[pl.ds(base, KE)], sidx)
            pltpu.sync_copy(dst_h.at[pl.ds(base, KE)], didx)
            cp1 = pltpu.async_copy(h2a_h.at[sidx], hbuf, sem1)
            cp2 = pltpu.async_copy(adst_h.at[didx], abuf, sem2)
            cp1.wait()
            cp2.wait()

            def edge(e, _):
                s = hbuf[e, pl.ds(HID, LANES)] + abuf[e, pl.ds(0, LANES)]
                ev = jnp.exp(jnp.maximum(s, 0.2 * s))
                for r in range(4):
                    sl = pl.ds(r * LANES, LANES)
                    obufs[r][e, pl.ds(0, LANES)] = hbuf[e, sl] * ev
                obufs[4][e, pl.ds(0, LANES)] = ev
                return 0

            lax.fori_loop(0, KE, edge, 0)
            for r in range(5):
                for j in range(KE // LANES):
                    sl = pl.ds(j * LANES, LANES)
                    d5[sl] = didx[sl] * 5 + r
                pltpu.sync_copy(obufs[r], oacc.at[d5], add=True)
            return 0

        lax.fori_loop(0, NBLK, blk, 0)
        plsc.subcore_barrier()
        pltpu.sync_copy(oacc.at[pl.ds(sid * RPT * 5, RPT * 5)],
                        acc_h.at[pl.ds((cid * NPAD + sid * RPT) * 5, RPT * 5)])

    acc2 = sc_b2(h2a, adst2t, src, dst, z80.reshape(RPT * 5, LANES))

    outf, _, _ = pl.pallas_call(
        _tc3_body,
        grid=(NB,),
        in_specs=[
            pl.BlockSpec((2, BN, 80), lambda i: (0, i, 0)),
            pl.BlockSpec((BN, 1), lambda i: (i, 0)),
            pl.BlockSpec((1, HID), lambda i: (0, 0)),
            pl.BlockSpec((HID, blin.shape[0]), lambda i: (0, 0)),
            pl.BlockSpec((1, blin.shape[0]), lambda i: (0, 0)),
        ],
        out_specs=[
            pl.BlockSpec((NG, blin.shape[0]), lambda i: (0, 0)),
            pl.BlockSpec((NG, HID), lambda i: (0, 0)),
            pl.BlockSpec((NG, HID), lambda i: (0, 0)),
        ],
        out_shape=[
            jax.ShapeDtypeStruct((NG, blin.shape[0]), f32),
            jax.ShapeDtypeStruct((NG, HID), f32),
            jax.ShapeDtypeStruct((NG, HID), f32),
        ],
    )(acc2.reshape(2, NPAD, 80), batch_p, b2r, Wlin, blr)

    return {"hc": hc, "asrc16": asrc16, "adst16": adst16, "ex": ex,
            "den2": den2, "outc": outc, "h2a": h2a, "adst2t": adst2t,
            "acc2": acc2, "outf": outf}


def kernel(x, edge_index, batch, W1, att_src1, att_dst1, b1,
           W2, att_src2, att_dst2, b2, Wlin, blin):
    return _pipeline(x, edge_index, batch, W1, att_src1, att_dst1, b1,
                     W2, att_src2, att_dst2, b2, Wlin, blin)["outf"]


# speedup vs baseline: 20.9343x; 1.5650x over previous
import functools

import jax
import jax.numpy as jnp
from jax import lax
from jax.experimental import pallas as pl
from jax.experimental.pallas import tpu as pltpu
from jax.experimental.pallas import tpu_sc as plsc

HEADS = 8
HID = 64
NG = 16
LANES = 16
NW = 32
KE = 128


def _tc1_body(x_ref, w_ref, as_ref, ad_ref, h_ref, asrc_ref, adst_ref):
    c = pl.program_id(1)
    h = jnp.dot(x_ref[...], w_ref[...], preferred_element_type=jnp.float32)
    h_ref[0] = h

    @pl.when(c == 0)
    def _():
        asrc_ref[...] = jnp.zeros_like(asrc_ref)
        adst_ref[...] = jnp.zeros_like(adst_ref)

    asrc_ref[...] += jnp.dot(h, as_ref[...], preferred_element_type=jnp.float32)
    adst_ref[...] += jnp.dot(h, ad_ref[...], preferred_element_type=jnp.float32)


def _tc2_body(outc_ref, den_ref, b1_ref, w2_ref, as2_ref, ad2_ref, e2_ref,
              h2a_ref, adst2_ref):
    chunks = []
    for c in range(4):
        raw = outc_ref[c]
        d = den_ref[0, :, 2 * c:2 * c + 2] + den_ref[1, :, 2 * c:2 * c + 2]
        db = jnp.dot(d + 1e-16, e2_ref[...],
                     preferred_element_type=jnp.float32)
        z = raw / db + b1_ref[0, c * 128:(c + 1) * 128]
        chunks.append(jnp.where(z > 0, z, jnp.exp(z) - 1.0))
    h1 = jnp.concatenate(chunks, axis=1)
    h2 = jnp.dot(h1, w2_ref[...], preferred_element_type=jnp.float32)
    a2s = jnp.dot(h2, as2_ref[...], preferred_element_type=jnp.float32)
    a2d = jnp.dot(h2, ad2_ref[...], preferred_element_type=jnp.float32)
    bn = h2.shape[0]
    h2a_ref[...] = jnp.concatenate(
        [h2, jnp.broadcast_to(a2s, (bn, LANES))], axis=1)
    adst2_ref[...] = jnp.broadcast_to(a2d, (bn, LANES))


def _tc3_body(acc_ref, batch_ref, b2_ref, wl_ref, bl_ref,
              out_ref, sums_ref, cnt_ref):
    i = pl.program_id(0)
    rows = acc_ref[0] + acc_ref[1]
    den = rows[:, HID:HID + 1] + 1e-16
    h2o = rows[:, :HID] / den + b2_ref[...]
    bn = h2o.shape[0]
    oh = (batch_ref[...] == lax.broadcasted_iota(jnp.int32, (1, NG), 1)
          ).astype(jnp.float32)

    @pl.when(i == 0)
    def _():
        sums_ref[...] = jnp.zeros_like(sums_ref)
        cnt_ref[...] = jnp.zeros_like(cnt_ref)

    dn = (((0,), (0,)), ((), ()))
    sums_ref[...] += lax.dot_general(oh, h2o, dn,
                                     preferred_element_type=jnp.float32)
    cnt_ref[...] += lax.dot_general(oh, jnp.ones((bn, HID), jnp.float32), dn,
                                    preferred_element_type=jnp.float32)
    pooled = sums_ref[...] / jnp.maximum(cnt_ref[...], 1.0)
    out_ref[...] = (jnp.dot(pooled, wl_ref[...],
                            preferred_element_type=jnp.float32) + bl_ref[...])


def _pipeline(x, edge_index, batch, W1, att_src1, att_dst1, b1,
              W2, att_src2, att_dst2, b2, Wlin, blin):
    f32, i32 = jnp.float32, jnp.int32
    N, F = x.shape
    E0 = edge_index.shape[1]
    ei = edge_index.astype(i32)

    NPAD = ((N + 255) // 256) * 256
    RPT = NPAD // 16
    E1 = E0 + N
    EPT = ((E1 + NW * KE - 1) // (NW * KE)) * KE
    NBLK = EPT // KE
    EPAD = EPT * NW

    loops = jnp.arange(N, dtype=i32)
    src = jnp.concatenate([ei[0], loops,
                           jnp.zeros((EPAD - E1,), i32)])
    dst = jnp.concatenate([ei[1], loops,
                           jnp.full((EPAD - E1,), N, i32)])
    xp = jnp.pad(x, ((0, NPAD - N), (0, 0)))
    batch_p = jnp.pad(batch.astype(i32), (0, NPAD - N),
                      constant_values=NG).reshape(NPAD, 1)

    eyeH = jnp.eye(HEADS, dtype=f32)
    A2s = (att_src1[0][:, :, None] * eyeH[:, None, :]).reshape(HEADS * HID, HEADS)
    A2d = (att_dst1[0][:, :, None] * eyeH[:, None, :]).reshape(HEADS * HID, HEADS)
    A2s = jnp.concatenate([A2s, A2s], axis=1)
    A2d = jnp.concatenate([A2d, A2d], axis=1)
    E2 = jnp.repeat(jnp.eye(2, dtype=f32), 128 // 2, axis=1)
    as2 = att_src2[0, 0].reshape(HID, 1)
    ad2 = att_dst2[0, 0].reshape(HID, 1)
    b1r = b1.reshape(1, HEADS * HID)
    b2r = b2.reshape(1, HID)
    blr = blin.reshape(1, -1)
    z16 = jnp.zeros((RPT, LANES), f32)
    z128 = jnp.zeros((RPT, 128), f32)
    z80 = jnp.zeros((RPT, 80), f32)

    BN = 256
    NB = NPAD // BN

    hc, asrc16, adst16 = pl.pallas_call(
        _tc1_body,
        grid=(NB, 4),
        in_specs=[
            pl.BlockSpec((BN, F), lambda i, c: (i, 0)),
            pl.BlockSpec((F, 128), lambda i, c: (0, c)),
            pl.BlockSpec((128, LANES), lambda i, c: (c, 0)),
            pl.BlockSpec((128, LANES), lambda i, c: (c, 0)),
        ],
        out_specs=[
            pl.BlockSpec((1, BN, 128), lambda i, c: (c, i, 0)),
            pl.BlockSpec((BN, LANES), lambda i, c: (i, 0)),
            pl.BlockSpec((BN, LANES), lambda i, c: (i, 0)),
        ],
        out_shape=[
            jax.ShapeDtypeStruct((4, NPAD, 128), f32),
            jax.ShapeDtypeStruct((NPAD, LANES), f32),
            jax.ShapeDtypeStruct((NPAD, LANES), f32),
        ],
    )(xp, W1, A2s, A2d)

    mesh = plsc.VectorSubcoreMesh(core_axis_name="c", subcore_axis_name="s")

    @functools.partial(
        pl.kernel,
        out_type=(jax.ShapeDtypeStruct((EPAD, LANES), f32),
                  jax.ShapeDtypeStruct((2 * NPAD, LANES), f32)),
        mesh=mesh,
        compiler_params=pltpu.CompilerParams(use_tc_tiling_on_sc=False),
        scratch_types=[
            pltpu.VMEM((KE,), i32), pltpu.VMEM((KE,), i32),
            pltpu.VMEM((KE, LANES), f32), pltpu.VMEM((KE, LANES), f32),
            pltpu.VMEM((KE, LANES), f32),
            pltpu.VMEM_SHARED((NPAD, LANES), f32),
            pltpu.SemaphoreType.DMA, pltpu.SemaphoreType.DMA,
        ],
    )
    def sc_a1(asrc_h, adst_h, src_h, dst_h, z16_h, ex_h, den_h,
              sidx, didx, asb, adb, exb, dacc, sem1, sem2):
        cid = lax.axis_index("c")
        sid = lax.axis_index("s")
        wid = cid * 16 + sid
        pltpu.sync_copy(z16_h, dacc.at[pl.ds(sid * RPT, RPT)])
        plsc.subcore_barrier()

        def blk(b, _):
            base = wid * EPT + b * KE
            pltpu.sync_copy(src_h.at[pl.ds(base, KE)], sidx)
            pltpu.sync_copy(dst_h.at[pl.ds(base, KE)], didx)
            cp1 = pltpu.async_copy(asrc_h.at[sidx], asb, sem1)
            cp2 = pltpu.async_copy(adst_h.at[didx], adb, sem2)
            cp1.wait()
            cp2.wait()

            def edge(e, _):
                s = asb[e, pl.ds(0, LANES)] + adb[e, pl.ds(0, LANES)]
                ev = jnp.exp(jnp.maximum(s, 0.2 * s))
                exb[e, pl.ds(0, LANES)] = ev
                return 0

            lax.fori_loop(0, KE, edge, 0)
            pltpu.sync_copy(exb, ex_h.at[pl.ds(base, KE)])
            pltpu.sync_copy(exb, dacc.at[didx], add=True)
            return 0

        lax.fori_loop(0, NBLK, blk, 0)
        plsc.subcore_barrier()
        pltpu.sync_copy(dacc.at[pl.ds(sid * RPT, RPT)],
                        den_h.at[pl.ds(cid * NPAD + sid * RPT, RPT)])

    ex, den2 = sc_a1(asrc16, adst16, src, dst, z16)

    EPT16 = EPAD // 16
    NBLK16 = EPT16 // KE
    @functools.partial(
        pl.kernel,
        out_type=jax.ShapeDtypeStruct((4 * NPAD, 128), f32),
        mesh=mesh,
        compiler_params=pltpu.CompilerParams(use_tc_tiling_on_sc=False),
        scratch_types=[
            pltpu.VMEM((KE,), i32), pltpu.VMEM((KE,), i32),
            pltpu.VMEM((KE,), i32),
            pltpu.VMEM((KE, 128), f32),
            pltpu.VMEM((KE, 128), f32),
            pltpu.VMEM((KE, LANES), f32),
            pltpu.VMEM_SHARED((NPAD, 128), f32),
            pltpu.SemaphoreType.DMA,
        ],
    )
    def sc_b1(hcat_h, exf_h, src4_h, dst_h, z128_h, outc_h,
              sidx, didx, d8, hbuf, sbuf, exb, oacc, sem):
        cid = lax.axis_index("c")
        sid = lax.axis_index("s")
        for chunk in range(2):
            c = cid * 2 + chunk
            cN = c * NPAD
            c2 = c * 2
            pltpu.sync_copy(z128_h, oacc.at[pl.ds(sid * RPT, RPT)])
            plsc.subcore_barrier()

            def blk(b, _):
                base = sid * EPT16 + b * KE
                pltpu.sync_copy(src4_h.at[pl.ds(c * EPAD + base, KE)], sidx)
                pltpu.sync_copy(dst_h.at[pl.ds(base, KE)], didx)
                pltpu.sync_copy(exf_h.at[pl.ds(base, KE)], exb)
                pltpu.async_copy(hcat_h.at[sidx], hbuf, sem).wait()

                def edge(e, _):
                    row = exb[e, pl.ds(0, LANES)]
                    s0 = row.at[jnp.full((LANES,), c2, i32)].get(
                        mode="promise_in_bounds")
                    s1 = row.at[jnp.full((LANES,), c2 + 1, i32)].get(
                        mode="promise_in_bounds")
                    for r in range(8):
                        sv = s0 if r < 4 else s1
                        sl = pl.ds(r * LANES, LANES)
                        sbuf[e, sl] = hbuf[e, sl] * sv
                    return 0

                lax.fori_loop(0, KE, edge, 0)
                pltpu.sync_copy(sbuf, oacc.at[didx], add=True)
                return 0

            lax.fori_loop(0, NBLK16, blk, 0)
            plsc.subcore_barrier()
            pltpu.sync_copy(oacc.at[pl.ds(sid * RPT, RPT)],
                            outc_h.at[pl.ds(cN + sid * RPT, RPT)])

    src4 = jnp.concatenate([src + c * NPAD for c in range(4)])
    outc = sc_b1(hc.reshape(4 * NPAD, 128), ex, src4, dst, z128)

    h2a, adst2t = pl.pallas_call(
        _tc2_body,
        grid=(NB,),
        in_specs=[
            pl.BlockSpec((4, BN, 128), lambda i: (0, i, 0)),
            pl.BlockSpec((2, BN, LANES), lambda i: (0, i, 0)),
            pl.BlockSpec((1, HEADS * HID), lambda i: (0, 0)),
            pl.BlockSpec((HEADS * HID, HID), lambda i: (0, 0)),
            pl.BlockSpec((HID, 1), lambda i: (0, 0)),
            pl.BlockSpec((HID, 1), lambda i: (0, 0)),
            pl.BlockSpec((2, 128), lambda i: (0, 0)),
        ],
        out_specs=[
            pl.BlockSpec((BN, HID + LANES), lambda i: (i, 0)),
            pl.BlockSpec((BN, LANES), lambda i: (i, 0)),
        ],
        out_shape=[
            jax.ShapeDtypeStruct((NPAD, HID + LANES), f32),
            jax.ShapeDtypeStruct((NPAD, LANES), f32),
        ],
    )(outc.reshape(4, NPAD, 128), den2.reshape(2, NPAD, LANES), b1r, W2,
      as2, ad2, E2)

    @functools.partial(
        pl.kernel,
        out_type=jax.ShapeDtypeStruct((2 * NPAD, 80), f32),
        mesh=mesh,
        compiler_params=pltpu.CompilerParams(use_tc_tiling_on_sc=False),
        scratch_types=[
            pltpu.VMEM((KE,), i32), pltpu.VMEM((KE,), i32),
            pltpu.VMEM((KE,), i32),
            pltpu.VMEM((KE, 80), f32), pltpu.VMEM((KE, LANES), f32),
            pltpu.VMEM((KE, 80), f32),
            pltpu.VMEM_SHARED((NPAD, 80), f32),
            pltpu.SemaphoreType.DMA, pltpu.SemaphoreType.DMA,
        ],
    )
    def sc_b2(h2a_h, adst_h, src_h, dst_h, z80_h, acc_h,
              sidx, didx, d5, hbuf, abuf, obuf, oacc, sem1, sem2):
        cid = lax.axis_index("c")
        sid = lax.axis_index("s")
        wid = cid * 16 + sid
        pltpu.sync_copy(z80_h, oacc.at[pl.ds(sid * RPT, RPT)])
        plsc.subcore_barrier()

        def blk(b, _):
            base = wid * EPT + b * KE
            pltpu.sync_copy(src_h.at[pl.ds(base, KE)], sidx)
            pltpu.sync_copy(dst_h.at[pl.ds(base, KE)], didx)
            cp1 = pltpu.async_copy(h2a_h.at[sidx], hbuf, sem1)
            cp2 = pltpu.async_copy(adst_h.at[didx], abuf, sem2)
            cp1.wait()
            cp2.wait()

            def edge(e, _):
                s = hbuf[e, pl.ds(HID, LANES)] + abuf[e, pl.ds(0, LANES)]
                ev = jnp.exp(jnp.maximum(s, 0.2 * s))
                for r in range(4):
                    sl = pl.ds(r * LANES, LANES)
                    obuf[e, sl] = hbuf[e, sl] * ev
                obuf[e, pl.ds(HID, LANES)] = ev
                return 0

            lax.fori_loop(0, KE, edge, 0)
            pltpu.sync_copy(obuf, oacc.at[didx], add=True)
            return 0

        lax.fori_loop(0, NBLK, blk, 0)
        plsc.subcore_barrier()
        pltpu.sync_copy(oacc.at[pl.ds(sid * RPT, RPT)],
                        acc_h.at[pl.ds(cid * NPAD + sid * RPT, RPT)])

    acc2 = sc_b2(h2a, adst2t, src, dst, z80)

    outf, _, _ = pl.pallas_call(
        _tc3_body,
        grid=(NB,),
        in_specs=[
            pl.BlockSpec((2, BN, 80), lambda i: (0, i, 0)),
            pl.BlockSpec((BN, 1), lambda i: (i, 0)),
            pl.BlockSpec((1, HID), lambda i: (0, 0)),
            pl.BlockSpec((HID, blin.shape[0]), lambda i: (0, 0)),
            pl.BlockSpec((1, blin.shape[0]), lambda i: (0, 0)),
        ],
        out_specs=[
            pl.BlockSpec((NG, blin.shape[0]), lambda i: (0, 0)),
            pl.BlockSpec((NG, HID), lambda i: (0, 0)),
            pl.BlockSpec((NG, HID), lambda i: (0, 0)),
        ],
        out_shape=[
            jax.ShapeDtypeStruct((NG, blin.shape[0]), f32),
            jax.ShapeDtypeStruct((NG, HID), f32),
            jax.ShapeDtypeStruct((NG, HID), f32),
        ],
    )(acc2.reshape(2, NPAD, 80), batch_p, b2r, Wlin, blr)

    return {"hc": hc, "asrc16": asrc16, "adst16": adst16, "ex": ex,
            "den2": den2, "outc": outc, "h2a": h2a, "adst2t": adst2t,
            "acc2": acc2, "outf": outf}


def kernel(x, edge_index, batch, W1, att_src1, att_dst1, b1,
           W2, att_src2, att_dst2, b2, Wlin, blin):
    return _pipeline(x, edge_index, batch, W1, att_src1, att_dst1, b1,
                     W2, att_src2, att_dst2, b2, Wlin, blin)["outf"]


# speedup vs baseline: 30.9370x; 1.4778x over previous
import functools

import jax
import jax.numpy as jnp
from jax import lax
from jax.experimental import pallas as pl
from jax.experimental.pallas import tpu as pltpu
from jax.experimental.pallas import tpu_sc as plsc

HEADS = 8
HID = 64
NG = 16
LANES = 16
NW = 32
KE = 128


def _tc1_body(x_ref, w_ref, as_ref, ad_ref, h_ref, asrc_ref, adst_ref):
    c = pl.program_id(1)
    h = jnp.dot(x_ref[...], w_ref[...], preferred_element_type=jnp.float32)
    h_ref[0] = h

    @pl.when(c == 0)
    def _():
        asrc_ref[...] = jnp.zeros_like(asrc_ref)
        adst_ref[...] = jnp.zeros_like(adst_ref)

    asrc_ref[...] += jnp.dot(h, as_ref[...], preferred_element_type=jnp.float32)
    adst_ref[...] += jnp.dot(h, ad_ref[...], preferred_element_type=jnp.float32)


def _tc2_body(outc_ref, den_ref, b1_ref, w2_ref, as2_ref, ad2_ref, e2_ref,
              h2a_ref, adst2_ref):
    chunks = []
    for c in range(4):
        raw = outc_ref[c]
        d = den_ref[0, :, 2 * c:2 * c + 2] + den_ref[1, :, 2 * c:2 * c + 2]
        db = jnp.dot(d + 1e-16, e2_ref[...],
                     preferred_element_type=jnp.float32)
        z = raw / db + b1_ref[0, c * 128:(c + 1) * 128]
        chunks.append(jnp.where(z > 0, z, jnp.exp(z) - 1.0))
    h1 = jnp.concatenate(chunks, axis=1)
    h2 = jnp.dot(h1, w2_ref[...], preferred_element_type=jnp.float32)
    a2s = jnp.dot(h2, as2_ref[...], preferred_element_type=jnp.float32)
    a2d = jnp.dot(h2, ad2_ref[...], preferred_element_type=jnp.float32)
    bn = h2.shape[0]
    h2a_ref[...] = jnp.concatenate(
        [h2, jnp.broadcast_to(a2s, (bn, LANES))], axis=1)
    adst2_ref[...] = jnp.broadcast_to(a2d, (bn, LANES))


def _tc3_body(acc_ref, batch_ref, b2_ref, wl_ref, bl_ref,
              out_ref, sums_ref, cnt_ref):
    i = pl.program_id(0)
    rows = acc_ref[0] + acc_ref[1]
    den = rows[:, HID:HID + 1] + 1e-16
    h2o = rows[:, :HID] / den + b2_ref[...]
    bn = h2o.shape[0]
    oh = (batch_ref[...] == lax.broadcasted_iota(jnp.int32, (1, NG), 1)
          ).astype(jnp.float32)

    @pl.when(i == 0)
    def _():
        sums_ref[...] = jnp.zeros_like(sums_ref)
        cnt_ref[...] = jnp.zeros_like(cnt_ref)

    dn = (((0,), (0,)), ((), ()))
    sums_ref[...] += lax.dot_general(oh, h2o, dn,
                                     preferred_element_type=jnp.float32)
    cnt_ref[...] += lax.dot_general(oh, jnp.ones((bn, HID), jnp.float32), dn,
                                    preferred_element_type=jnp.float32)
    pooled = sums_ref[...] / jnp.maximum(cnt_ref[...], 1.0)
    out_ref[...] = (jnp.dot(pooled, wl_ref[...],
                            preferred_element_type=jnp.float32) + bl_ref[...])


def _pipeline(x, edge_index, batch, W1, att_src1, att_dst1, b1,
              W2, att_src2, att_dst2, b2, Wlin, blin):
    f32, i32 = jnp.float32, jnp.int32
    N, F = x.shape
    E0 = edge_index.shape[1]
    ei = edge_index.astype(i32)

    NPAD = ((N + 255) // 256) * 256
    RPT = NPAD // 16
    E1 = E0 + N
    EPT = ((E1 + NW * KE - 1) // (NW * KE)) * KE
    NBLK = EPT // KE
    EPAD = EPT * NW

    loops = jnp.arange(N, dtype=i32)
    src = jnp.concatenate([ei[0], loops,
                           jnp.zeros((EPAD - E1,), i32)])
    dst = jnp.concatenate([ei[1], loops,
                           jnp.full((EPAD - E1,), N, i32)])
    xp = jnp.pad(x, ((0, NPAD - N), (0, 0)))
    batch_p = jnp.pad(batch.astype(i32), (0, NPAD - N),
                      constant_values=NG).reshape(NPAD, 1)

    eyeH = jnp.eye(HEADS, dtype=f32)
    A2s = (att_src1[0][:, :, None] * eyeH[:, None, :]).reshape(HEADS * HID, HEADS)
    A2d = (att_dst1[0][:, :, None] * eyeH[:, None, :]).reshape(HEADS * HID, HEADS)
    A2s = jnp.concatenate([A2s, A2s], axis=1)
    A2d = jnp.concatenate([A2d, A2d], axis=1)
    E2 = jnp.repeat(jnp.eye(2, dtype=f32), 128 // 2, axis=1)
    as2 = att_src2[0, 0].reshape(HID, 1)
    ad2 = att_dst2[0, 0].reshape(HID, 1)
    b1r = b1.reshape(1, HEADS * HID)
    b2r = b2.reshape(1, HID)
    blr = blin.reshape(1, -1)
    z16 = jnp.zeros((RPT, LANES), f32)
    z128 = jnp.zeros((RPT, 128), f32)
    z80 = jnp.zeros((RPT, 80), f32)

    BN = 256
    NB = NPAD // BN

    hc, asrc16, adst16 = pl.pallas_call(
        _tc1_body,
        grid=(NB, 4),
        in_specs=[
            pl.BlockSpec((BN, F), lambda i, c: (i, 0)),
            pl.BlockSpec((F, 128), lambda i, c: (0, c)),
            pl.BlockSpec((128, LANES), lambda i, c: (c, 0)),
            pl.BlockSpec((128, LANES), lambda i, c: (c, 0)),
        ],
        out_specs=[
            pl.BlockSpec((1, BN, 128), lambda i, c: (c, i, 0)),
            pl.BlockSpec((BN, LANES), lambda i, c: (i, 0)),
            pl.BlockSpec((BN, LANES), lambda i, c: (i, 0)),
        ],
        out_shape=[
            jax.ShapeDtypeStruct((4, NPAD, 128), f32),
            jax.ShapeDtypeStruct((NPAD, LANES), f32),
            jax.ShapeDtypeStruct((NPAD, LANES), f32),
        ],
    )(xp, W1, A2s, A2d)

    mesh = plsc.VectorSubcoreMesh(core_axis_name="c", subcore_axis_name="s")

    @functools.partial(
        pl.kernel,
        out_type=(jax.ShapeDtypeStruct((EPAD, LANES), f32),
                  jax.ShapeDtypeStruct((2 * NPAD, LANES), f32)),
        mesh=mesh,
        compiler_params=pltpu.CompilerParams(use_tc_tiling_on_sc=False),
        scratch_types=[
            pltpu.VMEM((KE,), i32), pltpu.VMEM((KE,), i32),
            pltpu.VMEM((KE, LANES), f32), pltpu.VMEM((KE, LANES), f32),
            pltpu.VMEM((KE, LANES), f32),
            pltpu.VMEM_SHARED((NPAD, LANES), f32),
            pltpu.SemaphoreType.DMA, pltpu.SemaphoreType.DMA,
        ],
    )
    def sc_a1(asrc_h, adst_h, src_h, dst_h, z16_h, ex_h, den_h,
              sidx, didx, asb, adb, exb, dacc, sem1, sem2):
        cid = lax.axis_index("c")
        sid = lax.axis_index("s")
        wid = cid * 16 + sid
        pltpu.sync_copy(z16_h, dacc.at[pl.ds(sid * RPT, RPT)])
        plsc.subcore_barrier()

        def blk(b, _):
            base = wid * EPT + b * KE
            pltpu.sync_copy(src_h.at[pl.ds(base, KE)], sidx)
            pltpu.sync_copy(dst_h.at[pl.ds(base, KE)], didx)
            cp1 = pltpu.async_copy(asrc_h.at[sidx], asb, sem1)
            cp2 = pltpu.async_copy(adst_h.at[didx], adb, sem2)
            cp1.wait()
            cp2.wait()

            def edge(e, _):
                s = asb[e, pl.ds(0, LANES)] + adb[e, pl.ds(0, LANES)]
                ev = jnp.exp(jnp.maximum(s, 0.2 * s))
                exb[e, pl.ds(0, LANES)] = ev
                return 0

            lax.fori_loop(0, KE, edge, 0)
            pltpu.sync_copy(exb, ex_h.at[pl.ds(base, KE)])
            pltpu.sync_copy(exb, dacc.at[didx], add=True)
            return 0

        lax.fori_loop(0, NBLK, blk, 0)
        plsc.subcore_barrier()
        pltpu.sync_copy(dacc.at[pl.ds(sid * RPT, RPT)],
                        den_h.at[pl.ds(cid * NPAD + sid * RPT, RPT)])

    ex, den2 = sc_a1(asrc16, adst16, src, dst, z16)

    EPT16 = EPAD // 16
    NBLK16 = EPT16 // KE

    @functools.partial(
        pl.kernel,
        out_type=jax.ShapeDtypeStruct((4 * NPAD, 128), f32),
        mesh=mesh,
        compiler_params=pltpu.CompilerParams(use_tc_tiling_on_sc=False),
        scratch_types=[
            pltpu.VMEM((4, KE), i32), pltpu.VMEM((4, KE), i32),
            [pltpu.VMEM((KE, 128), f32)] * 2,
            [pltpu.VMEM((KE, LANES), f32)] * 2,
            pltpu.VMEM_SHARED((NPAD, 128), f32),
            [pltpu.SemaphoreType.DMA] * 2,
            [pltpu.SemaphoreType.DMA] * 2,
            [pltpu.SemaphoreType.DMA] * 2,
            [pltpu.SemaphoreType.DMA] * 2,
            [pltpu.SemaphoreType.DMA] * 2,
        ],
    )
    def sc_b1(hcat_h, exf_h, src4_h, dst_h, z128_h, outc_h,
              sidxa, didxa, hbuf, exb, oacc,
              gsem, esem, ssem, ism, idm):
        cid = lax.axis_index("c")
        sid = lax.axis_index("s")

        def s_wait(s):
            pltpu.make_async_copy(hbuf[s], oacc.at[didxa.at[0]],
                                  ssem[s]).wait()

        for chunk in range(2):
            c = cid * 2 + chunk
            cN = c * NPAD
            c2 = c * 2
            ebase = sid * EPT16
            sbase = c * EPAD + ebase
            pltpu.sync_copy(z128_h, oacc.at[pl.ds(sid * RPT, RPT)])
            plsc.subcore_barrier()
            for b0 in range(2):
                pltpu.sync_copy(src4_h.at[pl.ds(sbase + b0 * KE, KE)],
                                sidxa.at[b0])
                pltpu.sync_copy(dst_h.at[pl.ds(ebase + b0 * KE, KE)],
                                didxa.at[b0])
            pltpu.async_copy(hcat_h.at[sidxa.at[0]], hbuf[0], gsem[0])
            pltpu.async_copy(exf_h.at[pl.ds(ebase, KE)], exb[0], esem[0])

            def blk2(t, _):
                for ph in range(2):
                    b = 2 * t + ph
                    cur, nxt = ph, 1 - ph

                    @pl.when(b >= 1)
                    def _():
                        s_wait(nxt)

                    @pl.when(b + 2 < NBLK16)
                    def _():
                        pltpu.async_copy(
                            src4_h.at[pl.ds(sbase + (b + 2) * KE, KE)],
                            sidxa.at[(b + 2) & 3], ism[cur])
                        pltpu.async_copy(
                            dst_h.at[pl.ds(ebase + (b + 2) * KE, KE)],
                            didxa.at[(b + 2) & 3], idm[cur])

                    @pl.when(b + 1 < NBLK16)
                    def _():
                        @pl.when(b + 1 >= 2)
                        def _():
                            pltpu.make_async_copy(
                                src4_h.at[pl.ds(sbase, KE)],
                                sidxa.at[0], ism[nxt]).wait()
                            pltpu.make_async_copy(
                                dst_h.at[pl.ds(ebase, KE)],
                                didxa.at[0], idm[nxt]).wait()

                        pltpu.async_copy(hcat_h.at[sidxa.at[(b + 1) & 3]],
                                         hbuf[nxt], gsem[nxt])
                        pltpu.async_copy(
                            exf_h.at[pl.ds(ebase + (b + 1) * KE, KE)],
                            exb[nxt], esem[nxt])

                    pltpu.make_async_copy(hcat_h.at[sidxa.at[0]], hbuf[cur],
                                          gsem[cur]).wait()
                    pltpu.make_async_copy(exf_h.at[pl.ds(0, KE)], exb[cur],
                                          esem[cur]).wait()

                    def edge(e, _):
                        row = exb[cur][e, pl.ds(0, LANES)]
                        s0 = row.at[jnp.full((LANES,), c2, i32)].get(
                            mode="promise_in_bounds")
                        s1 = row.at[jnp.full((LANES,), c2 + 1, i32)].get(
                            mode="promise_in_bounds")
                        for r in range(8):
                            sv = s0 if r < 4 else s1
                            sl = pl.ds(r * LANES, LANES)
                            hbuf[cur][e, sl] = hbuf[cur][e, sl] * sv
                        return 0

                    lax.fori_loop(0, KE, edge, 0)
                    pltpu.async_copy(hbuf[cur], oacc.at[didxa.at[b & 3]],
                                     ssem[cur], add=True)
                return 0

            lax.fori_loop(0, NBLK16 // 2, blk2, 0)
            s_wait((NBLK16 - 1) % 2)
            plsc.subcore_barrier()
            pltpu.sync_copy(oacc.at[pl.ds(sid * RPT, RPT)],
                            outc_h.at[pl.ds(cN + sid * RPT, RPT)])

    src4 = jnp.concatenate([src + c * NPAD for c in range(4)])
    outc = sc_b1(hc.reshape(4 * NPAD, 128), ex, src4, dst, z128)

    h2a, adst2t = pl.pallas_call(
        _tc2_body,
        grid=(NB,),
        in_specs=[
            pl.BlockSpec((4, BN, 128), lambda i: (0, i, 0)),
            pl.BlockSpec((2, BN, LANES), lambda i: (0, i, 0)),
            pl.BlockSpec((1, HEADS * HID), lambda i: (0, 0)),
            pl.BlockSpec((HEADS * HID, HID), lambda i: (0, 0)),
            pl.BlockSpec((HID, 1), lambda i: (0, 0)),
            pl.BlockSpec((HID, 1), lambda i: (0, 0)),
            pl.BlockSpec((2, 128), lambda i: (0, 0)),
        ],
        out_specs=[
            pl.BlockSpec((BN, HID + LANES), lambda i: (i, 0)),
            pl.BlockSpec((BN, LANES), lambda i: (i, 0)),
        ],
        out_shape=[
            jax.ShapeDtypeStruct((NPAD, HID + LANES), f32),
            jax.ShapeDtypeStruct((NPAD, LANES), f32),
        ],
    )(outc.reshape(4, NPAD, 128), den2.reshape(2, NPAD, LANES), b1r, W2,
      as2, ad2, E2)

    @functools.partial(
        pl.kernel,
        out_type=jax.ShapeDtypeStruct((2 * NPAD, 80), f32),
        mesh=mesh,
        compiler_params=pltpu.CompilerParams(use_tc_tiling_on_sc=False),
        scratch_types=[
            pltpu.VMEM((KE,), i32), pltpu.VMEM((KE,), i32),
            pltpu.VMEM((KE,), i32),
            pltpu.VMEM((KE, 80), f32), pltpu.VMEM((KE, LANES), f32),
            pltpu.VMEM((KE, 80), f32),
            pltpu.VMEM_SHARED((NPAD, 80), f32),
            pltpu.SemaphoreType.DMA, pltpu.SemaphoreType.DMA,
        ],
    )
    def sc_b2(h2a_h, adst_h, src_h, dst_h, z80_h, acc_h,
              sidx, didx, d5, hbuf, abuf, obuf, oacc, sem1, sem2):
        cid = lax.axis_index("c")
        sid = lax.axis_index("s")
        wid = cid * 16 + sid
        pltpu.sync_copy(z80_h, oacc.at[pl.ds(sid * RPT, RPT)])
        plsc.subcore_barrier()

        def blk(b, _):
            base = wid * EPT + b * KE
            pltpu.sync_copy(src_h.at[pl.ds(base, KE)], sidx)
            pltpu.sync_copy(dst_h.at[pl.ds(base, KE)], didx)
            cp1 = pltpu.async_copy(h2a_h.at[sidx], hbuf, sem1)
            cp2 = pltpu.async_copy(adst_h.at[didx], abuf, sem2)
            cp1.wait()
            cp2.wait()

            def edge(e, _):
                s = hbuf[e, pl.ds(HID, LANES)] + abuf[e, pl.ds(0, LANES)]
                ev = jnp.exp(jnp.maximum(s, 0.2 * s))
                for r in range(4):
                    sl = pl.ds(r * LANES, LANES)
                    obuf[e, sl] = hbuf[e, sl] * ev
                obuf[e, pl.ds(HID, LANES)] = ev
                return 0

            lax.fori_loop(0, KE, edge, 0)
            pltpu.sync_copy(obuf, oacc.at[didx], add=True)
            return 0

        lax.fori_loop(0, NBLK, blk, 0)
        plsc.subcore_barrier()
        pltpu.sync_copy(oacc.at[pl.ds(sid * RPT, RPT)],
                        acc_h.at[pl.ds(cid * NPAD + sid * RPT, RPT)])

    acc2 = sc_b2(h2a, adst2t, src, dst, z80)

    outf, _, _ = pl.pallas_call(
        _tc3_body,
        grid=(NB,),
        in_specs=[
            pl.BlockSpec((2, BN, 80), lambda i: (0, i, 0)),
            pl.BlockSpec((BN, 1), lambda i: (i, 0)),
            pl.BlockSpec((1, HID), lambda i: (0, 0)),
            pl.BlockSpec((HID, blin.shape[0]), lambda i: (0, 0)),
            pl.BlockSpec((1, blin.shape[0]), lambda i: (0, 0)),
        ],
        out_specs=[
            pl.BlockSpec((NG, blin.shape[0]), lambda i: (0, 0)),
            pl.BlockSpec((NG, HID), lambda i: (0, 0)),
            pl.BlockSpec((NG, HID), lambda i: (0, 0)),
        ],
        out_shape=[
            jax.ShapeDtypeStruct((NG, blin.shape[0]), f32),
            jax.ShapeDtypeStruct((NG, HID), f32),
            jax.ShapeDtypeStruct((NG, HID), f32),
        ],
    )(acc2.reshape(2, NPAD, 80), batch_p, b2r, Wlin, blr)

    return {"hc": hc, "asrc16": asrc16, "adst16": adst16, "ex": ex,
            "den2": den2, "outc": outc, "h2a": h2a, "adst2t": adst2t,
            "acc2": acc2, "outf": outf}


def kernel(x, edge_index, batch, W1, att_src1, att_dst1, b1,
           W2, att_src2, att_dst2, b2, Wlin, blin):
    return _pipeline(x, edge_index, batch, W1, att_src1, att_dst1, b1,
                     W2, att_src2, att_dst2, b2, Wlin, blin)["outf"]


# speedup vs baseline: 34.2498x; 1.1071x over previous
import functools

import jax
import jax.numpy as jnp
from jax import lax
from jax.experimental import pallas as pl
from jax.experimental.pallas import tpu as pltpu
from jax.experimental.pallas import tpu_sc as plsc

HEADS = 8
HID = 64
NG = 16
LANES = 16
NW = 32
KE = 128


def _tc1_body(x_ref, w_ref, as_ref, ad_ref, h_ref, asrc_ref, adst_ref):
    c = pl.program_id(1)
    h = jnp.dot(x_ref[...], w_ref[...], preferred_element_type=jnp.float32)
    h_ref[0] = h

    @pl.when(c == 0)
    def _():
        asrc_ref[...] = jnp.zeros_like(asrc_ref)
        adst_ref[...] = jnp.zeros_like(adst_ref)

    asrc_ref[...] += jnp.dot(h, as_ref[...], preferred_element_type=jnp.float32)
    adst_ref[...] += jnp.dot(h, ad_ref[...], preferred_element_type=jnp.float32)


def _tc2_body(outc_ref, den_ref, b1_ref, w2_ref, as2_ref, ad2_ref, e2_ref,
              h2a_ref, adst2_ref):
    chunks = []
    for c in range(4):
        raw = outc_ref[c]
        d = den_ref[0, :, 2 * c:2 * c + 2] + den_ref[1, :, 2 * c:2 * c + 2]
        db = jnp.dot(d + 1e-16, e2_ref[...],
                     preferred_element_type=jnp.float32)
        z = raw / db + b1_ref[0, c * 128:(c + 1) * 128]
        chunks.append(jnp.where(z > 0, z, jnp.exp(z) - 1.0))
    h1 = jnp.concatenate(chunks, axis=1)
    h2 = jnp.dot(h1, w2_ref[...], preferred_element_type=jnp.float32)
    a2s = jnp.dot(h2, as2_ref[...], preferred_element_type=jnp.float32)
    a2d = jnp.dot(h2, ad2_ref[...], preferred_element_type=jnp.float32)
    bn = h2.shape[0]
    h2a_ref[...] = jnp.concatenate(
        [h2, jnp.broadcast_to(a2s, (bn, LANES))], axis=1)
    adst2_ref[...] = jnp.broadcast_to(a2d, (bn, LANES))


def _tc3_body(acc_ref, batch_ref, b2_ref, wl_ref, bl_ref,
              out_ref, sums_ref, cnt_ref):
    i = pl.program_id(0)
    rows = acc_ref[0] + acc_ref[1]
    den = rows[:, HID:HID + 1] + 1e-16
    h2o = rows[:, :HID] / den + b2_ref[...]
    bn = h2o.shape[0]
    oh = (batch_ref[...] == lax.broadcasted_iota(jnp.int32, (1, NG), 1)
          ).astype(jnp.float32)

    @pl.when(i == 0)
    def _():
        sums_ref[...] = jnp.zeros_like(sums_ref)
        cnt_ref[...] = jnp.zeros_like(cnt_ref)

    dn = (((0,), (0,)), ((), ()))
    sums_ref[...] += lax.dot_general(oh, h2o, dn,
                                     preferred_element_type=jnp.float32)
    cnt_ref[...] += lax.dot_general(oh, jnp.ones((bn, HID), jnp.float32), dn,
                                    preferred_element_type=jnp.float32)
    pooled = sums_ref[...] / jnp.maximum(cnt_ref[...], 1.0)
    out_ref[...] = (jnp.dot(pooled, wl_ref[...],
                            preferred_element_type=jnp.float32) + bl_ref[...])


def _pipeline(x, edge_index, batch, W1, att_src1, att_dst1, b1,
              W2, att_src2, att_dst2, b2, Wlin, blin):
    f32, i32 = jnp.float32, jnp.int32
    N, F = x.shape
    E0 = edge_index.shape[1]
    ei = edge_index.astype(i32)

    NPAD = ((N + 255) // 256) * 256
    RPT = NPAD // 16
    E1 = E0 + N
    EPT = ((E1 + NW * KE - 1) // (NW * KE)) * KE
    NBLK = EPT // KE
    EPAD = EPT * NW

    loops = jnp.arange(N, dtype=i32)
    src = jnp.concatenate([ei[0], loops,
                           jnp.zeros((EPAD - E1,), i32)])
    dst = jnp.concatenate([ei[1], loops,
                           jnp.full((EPAD - E1,), N, i32)])
    xp = jnp.pad(x, ((0, NPAD - N), (0, 0)))
    batch_p = jnp.pad(batch.astype(i32), (0, NPAD - N),
                      constant_values=NG).reshape(NPAD, 1)

    eyeH = jnp.eye(HEADS, dtype=f32)
    A2s = (att_src1[0][:, :, None] * eyeH[:, None, :]).reshape(HEADS * HID, HEADS)
    A2d = (att_dst1[0][:, :, None] * eyeH[:, None, :]).reshape(HEADS * HID, HEADS)
    A2s = jnp.concatenate([A2s, A2s], axis=1)
    A2d = jnp.concatenate([A2d, A2d], axis=1)
    E2 = jnp.repeat(jnp.eye(2, dtype=f32), 128 // 2, axis=1)
    as2 = att_src2[0, 0].reshape(HID, 1)
    ad2 = att_dst2[0, 0].reshape(HID, 1)
    b1r = b1.reshape(1, HEADS * HID)
    b2r = b2.reshape(1, HID)
    blr = blin.reshape(1, -1)
    z16 = jnp.zeros((RPT, LANES), f32)
    z128 = jnp.zeros((RPT, 128), f32)
    z80 = jnp.zeros((RPT, 80), f32)

    BN = 256
    NB = NPAD // BN

    hc, asrc16, adst16 = pl.pallas_call(
        _tc1_body,
        grid=(NB, 4),
        in_specs=[
            pl.BlockSpec((BN, F), lambda i, c: (i, 0)),
            pl.BlockSpec((F, 128), lambda i, c: (0, c)),
            pl.BlockSpec((128, LANES), lambda i, c: (c, 0)),
            pl.BlockSpec((128, LANES), lambda i, c: (c, 0)),
        ],
        out_specs=[
            pl.BlockSpec((1, BN, 128), lambda i, c: (c, i, 0)),
            pl.BlockSpec((BN, LANES), lambda i, c: (i, 0)),
            pl.BlockSpec((BN, LANES), lambda i, c: (i, 0)),
        ],
        out_shape=[
            jax.ShapeDtypeStruct((4, NPAD, 128), f32),
            jax.ShapeDtypeStruct((NPAD, LANES), f32),
            jax.ShapeDtypeStruct((NPAD, LANES), f32),
        ],
    )(xp, W1, A2s, A2d)

    mesh = plsc.VectorSubcoreMesh(core_axis_name="c", subcore_axis_name="s")

    @functools.partial(
        pl.kernel,
        out_type=(jax.ShapeDtypeStruct((EPAD, LANES), f32),
                  jax.ShapeDtypeStruct((2 * NPAD, LANES), f32)),
        mesh=mesh,
        compiler_params=pltpu.CompilerParams(use_tc_tiling_on_sc=False),
        scratch_types=[
            pltpu.VMEM((KE,), i32), pltpu.VMEM((KE,), i32),
            pltpu.VMEM((KE, LANES), f32), pltpu.VMEM((KE, LANES), f32),
            pltpu.VMEM((KE, LANES), f32),
            pltpu.VMEM_SHARED((NPAD, LANES), f32),
            pltpu.SemaphoreType.DMA, pltpu.SemaphoreType.DMA,
        ],
    )
    def sc_a1(asrc_h, adst_h, src_h, dst_h, z16_h, ex_h, den_h,
              sidx, didx, asb, adb, exb, dacc, sem1, sem2):
        cid = lax.axis_index("c")
        sid = lax.axis_index("s")
        wid = cid * 16 + sid
        pltpu.sync_copy(z16_h, dacc.at[pl.ds(sid * RPT, RPT)])
        plsc.subcore_barrier()

        def blk(b, _):
            base = wid * EPT + b * KE
            pltpu.sync_copy(src_h.at[pl.ds(base, KE)], sidx)
            pltpu.sync_copy(dst_h.at[pl.ds(base, KE)], didx)
            cp1 = pltpu.async_copy(asrc_h.at[sidx], asb, sem1)
            cp2 = pltpu.async_copy(adst_h.at[didx], adb, sem2)
            cp1.wait()
            cp2.wait()

            def edge(e, _):
                s = asb[e, pl.ds(0, LANES)] + adb[e, pl.ds(0, LANES)]
                ev = jnp.exp(jnp.maximum(s, 0.2 * s))
                exb[e, pl.ds(0, LANES)] = ev
                return 0

            lax.fori_loop(0, KE, edge, 0)
            pltpu.sync_copy(exb, ex_h.at[pl.ds(base, KE)])
            pltpu.sync_copy(exb, dacc.at[didx], add=True)
            return 0

        lax.fori_loop(0, NBLK, blk, 0)
        plsc.subcore_barrier()
        pltpu.sync_copy(dacc.at[pl.ds(sid * RPT, RPT)],
                        den_h.at[pl.ds(cid * NPAD + sid * RPT, RPT)])

    ex, den2 = sc_a1(asrc16, adst16, src, dst, z16)

    EPT16 = EPAD // 16
    NBLK16 = EPT16 // KE

    @functools.partial(
        pl.kernel,
        out_type=jax.ShapeDtypeStruct((4 * NPAD, 128), f32),
        mesh=mesh,
        compiler_params=pltpu.CompilerParams(use_tc_tiling_on_sc=False),
        scratch_types=[
            pltpu.VMEM((4, KE), i32), pltpu.VMEM((4, KE), i32),
            [pltpu.VMEM((KE, 128), f32)] * 2,
            [pltpu.VMEM((KE, LANES), f32)] * 2,
            pltpu.VMEM_SHARED((NPAD, 128), f32),
            [pltpu.SemaphoreType.DMA] * 2,
            [pltpu.SemaphoreType.DMA] * 2,
            [pltpu.SemaphoreType.DMA] * 2,
            [pltpu.SemaphoreType.DMA] * 2,
            [pltpu.SemaphoreType.DMA] * 2,
        ],
    )
    def sc_b1(hcat_h, exf_h, src4_h, dst_h, z128_h, outc_h,
              sidxa, didxa, hbuf, exb, oacc,
              gsem, esem, ssem, ism, idm):
        cid = lax.axis_index("c")
        sid = lax.axis_index("s")

        def s_wait(s):
            pltpu.make_async_copy(hbuf[s], oacc.at[didxa.at[0]],
                                  ssem[s]).wait()

        for chunk in range(2):
            c = cid * 2 + chunk
            cN = c * NPAD
            c2 = c * 2
            ebase = sid * EPT16
            sbase = c * EPAD + ebase
            pltpu.sync_copy(z128_h, oacc.at[pl.ds(sid * RPT, RPT)])
            plsc.subcore_barrier()
            for b0 in range(2):
                pltpu.sync_copy(src4_h.at[pl.ds(sbase + b0 * KE, KE)],
                                sidxa.at[b0])
                pltpu.sync_copy(dst_h.at[pl.ds(ebase + b0 * KE, KE)],
                                didxa.at[b0])
            pltpu.async_copy(hcat_h.at[sidxa.at[0]], hbuf[0], gsem[0])
            pltpu.async_copy(exf_h.at[pl.ds(ebase, KE)], exb[0], esem[0])

            def blk2(t, _):
                for ph in range(2):
                    b = 2 * t + ph
                    cur, nxt = ph, 1 - ph

                    @pl.when(b >= 1)
                    def _():
                        s_wait(nxt)

                    @pl.when(b + 2 < NBLK16)
                    def _():
                        pltpu.async_copy(
                            src4_h.at[pl.ds(sbase + (b + 2) * KE, KE)],
                            sidxa.at[(b + 2) & 3], ism[cur])
                        pltpu.async_copy(
                            dst_h.at[pl.ds(ebase + (b + 2) * KE, KE)],
                            didxa.at[(b + 2) & 3], idm[cur])

                    @pl.when(b + 1 < NBLK16)
                    def _():
                        @pl.when(b + 1 >= 2)
                        def _():
                            pltpu.make_async_copy(
                                src4_h.at[pl.ds(sbase, KE)],
                                sidxa.at[0], ism[nxt]).wait()
                            pltpu.make_async_copy(
                                dst_h.at[pl.ds(ebase, KE)],
                                didxa.at[0], idm[nxt]).wait()

                        pltpu.async_copy(hcat_h.at[sidxa.at[(b + 1) & 3]],
                                         hbuf[nxt], gsem[nxt])
                        pltpu.async_copy(
                            exf_h.at[pl.ds(ebase + (b + 1) * KE, KE)],
                            exb[nxt], esem[nxt])

                    pltpu.make_async_copy(hcat_h.at[sidxa.at[0]], hbuf[cur],
                                          gsem[cur]).wait()
                    pltpu.make_async_copy(exf_h.at[pl.ds(0, KE)], exb[cur],
                                          esem[cur]).wait()

                    def edge(e, _):
                        row = exb[cur][e, pl.ds(0, LANES)]
                        s0 = row.at[jnp.full((LANES,), c2, i32)].get(
                            mode="promise_in_bounds")
                        s1 = row.at[jnp.full((LANES,), c2 + 1, i32)].get(
                            mode="promise_in_bounds")
                        for r in range(8):
                            sv = s0 if r < 4 else s1
                            sl = pl.ds(r * LANES, LANES)
                            hbuf[cur][e, sl] = hbuf[cur][e, sl] * sv
                        return 0

                    lax.fori_loop(0, KE, edge, 0)
                    pltpu.async_copy(hbuf[cur], oacc.at[didxa.at[b & 3]],
                                     ssem[cur], add=True)
                return 0

            lax.fori_loop(0, NBLK16 // 2, blk2, 0)
            s_wait((NBLK16 - 1) % 2)
            plsc.subcore_barrier()
            pltpu.sync_copy(oacc.at[pl.ds(sid * RPT, RPT)],
                            outc_h.at[pl.ds(cN + sid * RPT, RPT)])

    src4 = jnp.concatenate([src + c * NPAD for c in range(4)])
    outc = sc_b1(hc.reshape(4 * NPAD, 128), ex, src4, dst, z128)

    h2a, adst2t = pl.pallas_call(
        _tc2_body,
        grid=(NB,),
        in_specs=[
            pl.BlockSpec((4, BN, 128), lambda i: (0, i, 0)),
            pl.BlockSpec((2, BN, LANES), lambda i: (0, i, 0)),
            pl.BlockSpec((1, HEADS * HID), lambda i: (0, 0)),
            pl.BlockSpec((HEADS * HID, HID), lambda i: (0, 0)),
            pl.BlockSpec((HID, 1), lambda i: (0, 0)),
            pl.BlockSpec((HID, 1), lambda i: (0, 0)),
            pl.BlockSpec((2, 128), lambda i: (0, 0)),
        ],
        out_specs=[
            pl.BlockSpec((BN, HID + LANES), lambda i: (i, 0)),
            pl.BlockSpec((BN, LANES), lambda i: (i, 0)),
        ],
        out_shape=[
            jax.ShapeDtypeStruct((NPAD, HID + LANES), f32),
            jax.ShapeDtypeStruct((NPAD, LANES), f32),
        ],
    )(outc.reshape(4, NPAD, 128), den2.reshape(2, NPAD, LANES), b1r, W2,
      as2, ad2, E2)

    @functools.partial(
        pl.kernel,
        out_type=jax.ShapeDtypeStruct((2 * NPAD, 80), f32),
        mesh=mesh,
        compiler_params=pltpu.CompilerParams(use_tc_tiling_on_sc=False),
        scratch_types=[
            pltpu.VMEM((4, KE), i32), pltpu.VMEM((4, KE), i32),
            [pltpu.VMEM((KE, 80), f32)] * 2,
            [pltpu.VMEM((KE, LANES), f32)] * 2,
            pltpu.VMEM_SHARED((NPAD, 80), f32),
            [pltpu.SemaphoreType.DMA] * 2,
            [pltpu.SemaphoreType.DMA] * 2,
            [pltpu.SemaphoreType.DMA] * 2,
            [pltpu.SemaphoreType.DMA] * 2,
            [pltpu.SemaphoreType.DMA] * 2,
        ],
    )
    def sc_b2(h2a_h, adst_h, src_h, dst_h, z80_h, acc_h,
              sidxa, didxa, hbuf, abuf, oacc, gsem, asem, ssem, ism, idm):
        cid = lax.axis_index("c")
        sid = lax.axis_index("s")
        wid = cid * 16 + sid

        def s_wait(s):
            pltpu.make_async_copy(hbuf[s], oacc.at[didxa.at[0]],
                                  ssem[s]).wait()

        ebase = wid * EPT
        pltpu.sync_copy(z80_h, oacc.at[pl.ds(sid * RPT, RPT)])
        plsc.subcore_barrier()
        for b0 in range(2):
            pltpu.sync_copy(src_h.at[pl.ds(ebase + b0 * KE, KE)],
                            sidxa.at[b0])
            pltpu.sync_copy(dst_h.at[pl.ds(ebase + b0 * KE, KE)],
                            didxa.at[b0])
        pltpu.async_copy(h2a_h.at[sidxa.at[0]], hbuf[0], gsem[0])
        pltpu.async_copy(adst_h.at[didxa.at[0]], abuf[0], asem[0])

        def blk2(t, _):
            for ph in range(2):
                b = 2 * t + ph
                cur, nxt = ph, 1 - ph

                @pl.when(b >= 1)
                def _():
                    s_wait(nxt)

                @pl.when(b + 2 < NBLK)
                def _():
                    pltpu.async_copy(
                        src_h.at[pl.ds(ebase + (b + 2) * KE, KE)],
                        sidxa.at[(b + 2) & 3], ism[cur])
                    pltpu.async_copy(
                        dst_h.at[pl.ds(ebase + (b + 2) * KE, KE)],
                        didxa.at[(b + 2) & 3], idm[cur])

                @pl.when(b + 1 < NBLK)
                def _():
                    @pl.when(b + 1 >= 2)
                    def _():
                        pltpu.make_async_copy(
                            src_h.at[pl.ds(ebase, KE)],
                            sidxa.at[0], ism[nxt]).wait()
                        pltpu.make_async_copy(
                            dst_h.at[pl.ds(ebase, KE)],
                            didxa.at[0], idm[nxt]).wait()

                    pltpu.async_copy(h2a_h.at[sidxa.at[(b + 1) & 3]],
                                     hbuf[nxt], gsem[nxt])
                    pltpu.async_copy(adst_h.at[didxa.at[(b + 1) & 3]],
                                     abuf[nxt], asem[nxt])

                pltpu.make_async_copy(h2a_h.at[sidxa.at[0]], hbuf[cur],
                                      gsem[cur]).wait()
                pltpu.make_async_copy(adst_h.at[didxa.at[0]], abuf[cur],
                                      asem[cur]).wait()

                def edge(e, _):
                    s = (hbuf[cur][e, pl.ds(HID, LANES)]
                         + abuf[cur][e, pl.ds(0, LANES)])
                    ev = jnp.exp(jnp.maximum(s, 0.2 * s))
                    for r in range(4):
                        sl = pl.ds(r * LANES, LANES)
                        hbuf[cur][e, sl] = hbuf[cur][e, sl] * ev
                    hbuf[cur][e, pl.ds(HID, LANES)] = ev
                    return 0

                lax.fori_loop(0, KE, edge, 0)
                pltpu.async_copy(hbuf[cur], oacc.at[didxa.at[b & 3]],
                                 ssem[cur], add=True)
            return 0

        lax.fori_loop(0, NBLK // 2, blk2, 0)

        if NBLK % 2 == 1:
            b = NBLK - 1
            cur, nxt = 0, 1
            s_wait(nxt)
            pltpu.make_async_copy(h2a_h.at[sidxa.at[0]], hbuf[cur],
                                  gsem[cur]).wait()
            pltpu.make_async_copy(adst_h.at[didxa.at[0]], abuf[cur],
                                  asem[cur]).wait()

            def edge(e, _):
                s = (hbuf[cur][e, pl.ds(HID, LANES)]
                     + abuf[cur][e, pl.ds(0, LANES)])
                ev = jnp.exp(jnp.maximum(s, 0.2 * s))
                for r in range(4):
                    sl = pl.ds(r * LANES, LANES)
                    hbuf[cur][e, sl] = hbuf[cur][e, sl] * ev
                hbuf[cur][e, pl.ds(HID, LANES)] = ev
                return 0

            lax.fori_loop(0, KE, edge, 0)
            pltpu.async_copy(hbuf[cur], oacc.at[didxa.at[b & 3]],
                             ssem[cur], add=True)

        s_wait((NBLK - 1) % 2)
        plsc.subcore_barrier()
        pltpu.sync_copy(oacc.at[pl.ds(sid * RPT, RPT)],
                        acc_h.at[pl.ds(cid * NPAD + sid * RPT, RPT)])

    acc2 = sc_b2(h2a, adst2t, src, dst, z80)

    outf, _, _ = pl.pallas_call(
        _tc3_body,
        grid=(NB,),
        in_specs=[
            pl.BlockSpec((2, BN, 80), lambda i: (0, i, 0)),
            pl.BlockSpec((BN, 1), lambda i: (i, 0)),
            pl.BlockSpec((1, HID), lambda i: (0, 0)),
            pl.BlockSpec((HID, blin.shape[0]), lambda i: (0, 0)),
            pl.BlockSpec((1, blin.shape[0]), lambda i: (0, 0)),
        ],
        out_specs=[
            pl.BlockSpec((NG, blin.shape[0]), lambda i: (0, 0)),
            pl.BlockSpec((NG, HID), lambda i: (0, 0)),
            pl.BlockSpec((NG, HID), lambda i: (0, 0)),
        ],
        out_shape=[
            jax.ShapeDtypeStruct((NG, blin.shape[0]), f32),
            jax.ShapeDtypeStruct((NG, HID), f32),
            jax.ShapeDtypeStruct((NG, HID), f32),
        ],
    )(acc2.reshape(2, NPAD, 80), batch_p, b2r, Wlin, blr)

    return {"hc": hc, "asrc16": asrc16, "adst16": adst16, "ex": ex,
            "den2": den2, "outc": outc, "h2a": h2a, "adst2t": adst2t,
            "acc2": acc2, "outf": outf}


def kernel(x, edge_index, batch, W1, att_src1, att_dst1, b1,
           W2, att_src2, att_dst2, b2, Wlin, blin):
    return _pipeline(x, edge_index, batch, W1, att_src1, att_dst1, b1,
                     W2, att_src2, att_dst2, b2, Wlin, blin)["outf"]


# speedup vs baseline: 40.8821x; 1.1936x over previous
import functools

import jax
import jax.numpy as jnp
from jax import lax
from jax.experimental import pallas as pl
from jax.experimental.pallas import tpu as pltpu
from jax.experimental.pallas import tpu_sc as plsc

HEADS = 8
HID = 64
NG = 16
LANES = 16
NW = 32
KE = 128


def _tc1_body(x_ref, w_ref, as_ref, ad_ref, h_ref, asrc_ref, adst_ref):
    c = pl.program_id(1)
    h = jnp.dot(x_ref[...], w_ref[...], preferred_element_type=jnp.float32)
    h_ref[0] = h

    @pl.when(c == 0)
    def _():
        asrc_ref[...] = jnp.zeros_like(asrc_ref)
        adst_ref[...] = jnp.zeros_like(adst_ref)

    asrc_ref[...] += jnp.dot(h, as_ref[...], preferred_element_type=jnp.float32)
    adst_ref[...] += jnp.dot(h, ad_ref[...], preferred_element_type=jnp.float32)


def _tc2_body(outc_ref, den_ref, b1_ref, w2_ref, as2_ref, ad2_ref, e2_ref,
              h2a_ref, adst2_ref):
    chunks = []
    for c in range(4):
        raw = outc_ref[c]
        d = den_ref[0, :, 2 * c:2 * c + 2] + den_ref[1, :, 2 * c:2 * c + 2]
        db = jnp.dot(d + 1e-16, e2_ref[...],
                     preferred_element_type=jnp.float32)
        z = raw / db + b1_ref[0, c * 128:(c + 1) * 128]
        chunks.append(jnp.where(z > 0, z, jnp.exp(z) - 1.0))
    h1 = jnp.concatenate(chunks, axis=1)
    h2 = jnp.dot(h1, w2_ref[...], preferred_element_type=jnp.float32)
    a2s = jnp.dot(h2, as2_ref[...], preferred_element_type=jnp.float32)
    a2d = jnp.dot(h2, ad2_ref[...], preferred_element_type=jnp.float32)
    bn = h2.shape[0]
    h2a_ref[...] = jnp.concatenate(
        [h2, jnp.broadcast_to(a2s, (bn, LANES))], axis=1)
    adst2_ref[...] = jnp.broadcast_to(a2d, (bn, LANES))


def _tc3_body(acc_ref, batch_ref, b2_ref, wl_ref, bl_ref,
              out_ref, sums_ref, cnt_ref):
    i = pl.program_id(0)
    rows = acc_ref[0] + acc_ref[1]
    den = rows[:, HID:HID + 1] + 1e-16
    h2o = rows[:, :HID] / den + b2_ref[...]
    bn = h2o.shape[0]
    oh = (batch_ref[...] == lax.broadcasted_iota(jnp.int32, (1, NG), 1)
          ).astype(jnp.float32)

    @pl.when(i == 0)
    def _():
        sums_ref[...] = jnp.zeros_like(sums_ref)
        cnt_ref[...] = jnp.zeros_like(cnt_ref)

    dn = (((0,), (0,)), ((), ()))
    sums_ref[...] += lax.dot_general(oh, h2o, dn,
                                     preferred_element_type=jnp.float32)
    cnt_ref[...] += lax.dot_general(oh, jnp.ones((bn, HID), jnp.float32), dn,
                                    preferred_element_type=jnp.float32)
    pooled = sums_ref[...] / jnp.maximum(cnt_ref[...], 1.0)
    out_ref[...] = (jnp.dot(pooled, wl_ref[...],
                            preferred_element_type=jnp.float32) + bl_ref[...])


def _pipeline(x, edge_index, batch, W1, att_src1, att_dst1, b1,
              W2, att_src2, att_dst2, b2, Wlin, blin):
    f32, i32 = jnp.float32, jnp.int32
    N, F = x.shape
    E0 = edge_index.shape[1]
    ei = edge_index.astype(i32)

    NPAD = ((N + 255) // 256) * 256
    RPT = NPAD // 16
    E1 = E0 + N
    EPT = ((E1 + NW * KE - 1) // (NW * KE)) * KE
    NBLK = EPT // KE
    EPAD = EPT * NW

    loops = jnp.arange(N, dtype=i32)
    src = jnp.concatenate([ei[0], loops,
                           jnp.zeros((EPAD - E1,), i32)])
    dst = jnp.concatenate([ei[1], loops,
                           jnp.full((EPAD - E1,), N, i32)])
    xp = jnp.pad(x, ((0, NPAD - N), (0, 0)))
    batch_p = jnp.pad(batch.astype(i32), (0, NPAD - N),
                      constant_values=NG).reshape(NPAD, 1)

    eyeH = jnp.eye(HEADS, dtype=f32)
    A2s = (att_src1[0][:, :, None] * eyeH[:, None, :]).reshape(HEADS * HID, HEADS)
    A2d = (att_dst1[0][:, :, None] * eyeH[:, None, :]).reshape(HEADS * HID, HEADS)
    A2s = jnp.concatenate([A2s, A2s], axis=1)
    A2d = jnp.concatenate([A2d, A2d], axis=1)
    E2 = jnp.repeat(jnp.eye(2, dtype=f32), 128 // 2, axis=1)
    as2 = att_src2[0, 0].reshape(HID, 1)
    ad2 = att_dst2[0, 0].reshape(HID, 1)
    b1r = b1.reshape(1, HEADS * HID)
    b2r = b2.reshape(1, HID)
    blr = blin.reshape(1, -1)
    z16 = jnp.zeros((RPT, LANES), f32)
    z128 = jnp.zeros((RPT, 128), f32)
    z80 = jnp.zeros((RPT, 80), f32)

    BN = 256
    NB = NPAD // BN

    hc, asrc16, adst16 = pl.pallas_call(
        _tc1_body,
        grid=(NB, 4),
        in_specs=[
            pl.BlockSpec((BN, F), lambda i, c: (i, 0)),
            pl.BlockSpec((F, 128), lambda i, c: (0, c)),
            pl.BlockSpec((128, LANES), lambda i, c: (c, 0)),
            pl.BlockSpec((128, LANES), lambda i, c: (c, 0)),
        ],
        out_specs=[
            pl.BlockSpec((1, BN, 128), lambda i, c: (c, i, 0)),
            pl.BlockSpec((BN, LANES), lambda i, c: (i, 0)),
            pl.BlockSpec((BN, LANES), lambda i, c: (i, 0)),
        ],
        out_shape=[
            jax.ShapeDtypeStruct((4, NPAD, 128), f32),
            jax.ShapeDtypeStruct((NPAD, LANES), f32),
            jax.ShapeDtypeStruct((NPAD, LANES), f32),
        ],
    )(xp, W1, A2s, A2d)

    mesh = plsc.VectorSubcoreMesh(core_axis_name="c", subcore_axis_name="s")

    @functools.partial(
        pl.kernel,
        out_type=(jax.ShapeDtypeStruct((EPAD, LANES), f32),
                  jax.ShapeDtypeStruct((2 * NPAD, LANES), f32)),
        mesh=mesh,
        compiler_params=pltpu.CompilerParams(use_tc_tiling_on_sc=False),
        scratch_types=[
            pltpu.VMEM((KE,), i32), pltpu.VMEM((KE,), i32),
            pltpu.VMEM((KE, LANES), f32), pltpu.VMEM((KE, LANES), f32),
            pltpu.VMEM((KE, LANES), f32),
            pltpu.VMEM_SHARED((NPAD, LANES), f32),
            pltpu.SemaphoreType.DMA, pltpu.SemaphoreType.DMA,
        ],
    )
    def sc_a1(asrc_h, adst_h, src_h, dst_h, z16_h, ex_h, den_h,
              sidx, didx, asb, adb, exb, dacc, sem1, sem2):
        cid = lax.axis_index("c")
        sid = lax.axis_index("s")
        wid = cid * 16 + sid
        pltpu.sync_copy(z16_h, dacc.at[pl.ds(sid * RPT, RPT)])
        plsc.subcore_barrier()

        def blk(b, _):
            base = wid * EPT + b * KE
            pltpu.sync_copy(src_h.at[pl.ds(base, KE)], sidx)
            pltpu.sync_copy(dst_h.at[pl.ds(base, KE)], didx)
            cp1 = pltpu.async_copy(asrc_h.at[sidx], asb, sem1)
            cp2 = pltpu.async_copy(adst_h.at[didx], adb, sem2)
            cp1.wait()
            cp2.wait()

            def edge(e, _):
                s = asb[e, pl.ds(0, LANES)] + adb[e, pl.ds(0, LANES)]
                ev = jnp.exp(jnp.maximum(s, 0.2 * s))
                exb[e, pl.ds(0, LANES)] = ev
                return 0

            lax.fori_loop(0, KE, edge, 0)
            pltpu.sync_copy(exb, ex_h.at[pl.ds(base, KE)])
            pltpu.sync_copy(exb, dacc.at[didx], add=True)
            return 0

        lax.fori_loop(0, NBLK, blk, 0)
        plsc.subcore_barrier()
        pltpu.sync_copy(dacc.at[pl.ds(sid * RPT, RPT)],
                        den_h.at[pl.ds(cid * NPAD + sid * RPT, RPT)])

    ex, den2 = sc_a1(asrc16, adst16, src, dst, z16)

    EPT16 = EPAD // 16
    NBLK16 = EPT16 // KE

    @functools.partial(
        pl.kernel,
        out_type=jax.ShapeDtypeStruct((4 * NPAD, 128), f32),
        mesh=mesh,
        compiler_params=pltpu.CompilerParams(use_tc_tiling_on_sc=False),
        scratch_types=[
            pltpu.VMEM((4, KE), i32), pltpu.VMEM((4, KE), i32),
            [pltpu.VMEM((KE, 128), f32)] * 2,
            [pltpu.VMEM((KE, LANES), f32)] * 2,
            pltpu.VMEM_SHARED((NPAD, 128), f32),
            [pltpu.SemaphoreType.DMA] * 2,
            [pltpu.SemaphoreType.DMA] * 2,
            [pltpu.SemaphoreType.DMA] * 2,
            [pltpu.SemaphoreType.DMA] * 2,
            [pltpu.SemaphoreType.DMA] * 2,
        ],
    )
    def sc_b1(hcat_h, exf_h, src4_h, dst_h, z128_h, outc_h,
              sidxa, didxa, hbuf, exb, oacc,
              gsem, esem, ssem, ism, idm):
        cid = lax.axis_index("c")
        sid = lax.axis_index("s")

        def s_wait(s):
            pltpu.make_async_copy(hbuf[s], oacc.at[didxa.at[0]],
                                  ssem[s]).wait()

        for chunk in range(2):
            c = cid * 2 + chunk
            cN = c * NPAD
            c2 = c * 2
            ebase = sid * EPT16
            sbase = c * EPAD + ebase
            pltpu.sync_copy(z128_h, oacc.at[pl.ds(sid * RPT, RPT)])
            plsc.subcore_barrier()
            for b0 in range(2):
                pltpu.sync_copy(src4_h.at[pl.ds(sbase + b0 * KE, KE)],
                                sidxa.at[b0])
                pltpu.sync_copy(dst_h.at[pl.ds(ebase + b0 * KE, KE)],
                                didxa.at[b0])
            pltpu.async_copy(hcat_h.at[sidxa.at[0]], hbuf[0], gsem[0])
            pltpu.async_copy(exf_h.at[pl.ds(ebase, KE)], exb[0], esem[0])

            def blk2(t, _):
                for ph in range(2):
                    b = 2 * t + ph
                    cur, nxt = ph, 1 - ph

                    @pl.when(b >= 1)
                    def _():
                        s_wait(nxt)

                    @pl.when(b + 2 < NBLK16)
                    def _():
                        pltpu.async_copy(
                            src4_h.at[pl.ds(sbase + (b + 2) * KE, KE)],
                            sidxa.at[(b + 2) & 3], ism[cur])
                        pltpu.async_copy(
                            dst_h.at[pl.ds(ebase + (b + 2) * KE, KE)],
                            didxa.at[(b + 2) & 3], idm[cur])

                    @pl.when(b + 1 < NBLK16)
                    def _():
                        @pl.when(b + 1 >= 2)
                        def _():
                            pltpu.make_async_copy(
                                src4_h.at[pl.ds(sbase, KE)],
                                sidxa.at[0], ism[nxt]).wait()
                            pltpu.make_async_copy(
                                dst_h.at[pl.ds(ebase, KE)],
                                didxa.at[0], idm[nxt]).wait()

                        pltpu.async_copy(hcat_h.at[sidxa.at[(b + 1) & 3]],
                                         hbuf[nxt], gsem[nxt])
                        pltpu.async_copy(
                            exf_h.at[pl.ds(ebase + (b + 1) * KE, KE)],
                            exb[nxt], esem[nxt])

                    pltpu.make_async_copy(hcat_h.at[sidxa.at[0]], hbuf[cur],
                                          gsem[cur]).wait()
                    pltpu.make_async_copy(exf_h.at[pl.ds(0, KE)], exb[cur],
                                          esem[cur]).wait()

                    @plsc.parallel_loop(0, KE, 1, unroll=4)
                    def edge(e):
                        row = exb[cur][e, pl.ds(0, LANES)]
                        s0 = row.at[jnp.full((LANES,), c2, i32)].get(
                            mode="promise_in_bounds")
                        s1 = row.at[jnp.full((LANES,), c2 + 1, i32)].get(
                            mode="promise_in_bounds")
                        for r in range(8):
                            sv = s0 if r < 4 else s1
                            sl = pl.ds(r * LANES, LANES)
                            hbuf[cur][e, sl] = hbuf[cur][e, sl] * sv
                    pltpu.async_copy(hbuf[cur], oacc.at[didxa.at[b & 3]],
                                     ssem[cur], add=True)
                return 0

            lax.fori_loop(0, NBLK16 // 2, blk2, 0)
            s_wait((NBLK16 - 1) % 2)
            plsc.subcore_barrier()
            pltpu.sync_copy(oacc.at[pl.ds(sid * RPT, RPT)],
                            outc_h.at[pl.ds(cN + sid * RPT, RPT)])

    src4 = jnp.concatenate([src + c * NPAD for c in range(4)])
    outc = sc_b1(hc.reshape(4 * NPAD, 128), ex, src4, dst, z128)

    h2a, adst2t = pl.pallas_call(
        _tc2_body,
        grid=(NB,),
        in_specs=[
            pl.BlockSpec((4, BN, 128), lambda i: (0, i, 0)),
            pl.BlockSpec((2, BN, LANES), lambda i: (0, i, 0)),
            pl.BlockSpec((1, HEADS * HID), lambda i: (0, 0)),
            pl.BlockSpec((HEADS * HID, HID), lambda i: (0, 0)),
            pl.BlockSpec((HID, 1), lambda i: (0, 0)),
            pl.BlockSpec((HID, 1), lambda i: (0, 0)),
            pl.BlockSpec((2, 128), lambda i: (0, 0)),
        ],
        out_specs=[
            pl.BlockSpec((BN, HID + LANES), lambda i: (i, 0)),
            pl.BlockSpec((BN, LANES), lambda i: (i, 0)),
        ],
        out_shape=[
            jax.ShapeDtypeStruct((NPAD, HID + LANES), f32),
            jax.ShapeDtypeStruct((NPAD, LANES), f32),
        ],
    )(outc.reshape(4, NPAD, 128), den2.reshape(2, NPAD, LANES), b1r, W2,
      as2, ad2, E2)

    @functools.partial(
        pl.kernel,
        out_type=jax.ShapeDtypeStruct((2 * NPAD, 80), f32),
        mesh=mesh,
        compiler_params=pltpu.CompilerParams(use_tc_tiling_on_sc=False),
        scratch_types=[
            pltpu.VMEM((4, KE), i32), pltpu.VMEM((4, KE), i32),
            [pltpu.VMEM((KE, 80), f32)] * 2,
            [pltpu.VMEM((KE, LANES), f32)] * 2,
            pltpu.VMEM_SHARED((NPAD, 80), f32),
            [pltpu.SemaphoreType.DMA] * 2,
            [pltpu.SemaphoreType.DMA] * 2,
            [pltpu.SemaphoreType.DMA] * 2,
            [pltpu.SemaphoreType.DMA] * 2,
            [pltpu.SemaphoreType.DMA] * 2,
        ],
    )
    def sc_b2(h2a_h, adst_h, src_h, dst_h, z80_h, acc_h,
              sidxa, didxa, hbuf, abuf, oacc, gsem, asem, ssem, ism, idm):
        cid = lax.axis_index("c")
        sid = lax.axis_index("s")
        wid = cid * 16 + sid

        def s_wait(s):
            pltpu.make_async_copy(hbuf[s], oacc.at[didxa.at[0]],
                                  ssem[s]).wait()

        ebase = wid * EPT
        pltpu.sync_copy(z80_h, oacc.at[pl.ds(sid * RPT, RPT)])
        plsc.subcore_barrier()
        for b0 in range(2):
            pltpu.sync_copy(src_h.at[pl.ds(ebase + b0 * KE, KE)],
                            sidxa.at[b0])
            pltpu.sync_copy(dst_h.at[pl.ds(ebase + b0 * KE, KE)],
                            didxa.at[b0])
        pltpu.async_copy(h2a_h.at[sidxa.at[0]], hbuf[0], gsem[0])
        pltpu.async_copy(adst_h.at[didxa.at[0]], abuf[0], asem[0])

        def blk2(t, _):
            for ph in range(2):
                b = 2 * t + ph
                cur, nxt = ph, 1 - ph

                @pl.when(b >= 1)
                def _():
                    s_wait(nxt)

                @pl.when(b + 2 < NBLK)
                def _():
                    pltpu.async_copy(
                        src_h.at[pl.ds(ebase + (b + 2) * KE, KE)],
                        sidxa.at[(b + 2) & 3], ism[cur])
                    pltpu.async_copy(
                        dst_h.at[pl.ds(ebase + (b + 2) * KE, KE)],
                        didxa.at[(b + 2) & 3], idm[cur])

                @pl.when(b + 1 < NBLK)
                def _():
                    @pl.when(b + 1 >= 2)
                    def _():
                        pltpu.make_async_copy(
                            src_h.at[pl.ds(ebase, KE)],
                            sidxa.at[0], ism[nxt]).wait()
                        pltpu.make_async_copy(
                            dst_h.at[pl.ds(ebase, KE)],
                            didxa.at[0], idm[nxt]).wait()

                    pltpu.async_copy(h2a_h.at[sidxa.at[(b + 1) & 3]],
                                     hbuf[nxt], gsem[nxt])
                    pltpu.async_copy(adst_h.at[didxa.at[(b + 1) & 3]],
                                     abuf[nxt], asem[nxt])

                pltpu.make_async_copy(h2a_h.at[sidxa.at[0]], hbuf[cur],
                                      gsem[cur]).wait()
                pltpu.make_async_copy(adst_h.at[didxa.at[0]], abuf[cur],
                                      asem[cur]).wait()

                @plsc.parallel_loop(0, KE, 1, unroll=4)
                def edge(e):
                    s = (hbuf[cur][e, pl.ds(HID, LANES)]
                         + abuf[cur][e, pl.ds(0, LANES)])
                    ev = jnp.exp(jnp.maximum(s, 0.2 * s))
                    for r in range(4):
                        sl = pl.ds(r * LANES, LANES)
                        hbuf[cur][e, sl] = hbuf[cur][e, sl] * ev
                    hbuf[cur][e, pl.ds(HID, LANES)] = ev
                pltpu.async_copy(hbuf[cur], oacc.at[didxa.at[b & 3]],
                                 ssem[cur], add=True)
            return 0

        lax.fori_loop(0, NBLK // 2, blk2, 0)

        if NBLK % 2 == 1:
            b = NBLK - 1
            cur, nxt = 0, 1
            s_wait(nxt)
            pltpu.make_async_copy(h2a_h.at[sidxa.at[0]], hbuf[cur],
                                  gsem[cur]).wait()
            pltpu.make_async_copy(adst_h.at[didxa.at[0]], abuf[cur],
                                  asem[cur]).wait()

            @plsc.parallel_loop(0, KE, 1, unroll=4)
            def edge(e):
                s = (hbuf[cur][e, pl.ds(HID, LANES)]
                     + abuf[cur][e, pl.ds(0, LANES)])
                ev = jnp.exp(jnp.maximum(s, 0.2 * s))
                for r in range(4):
                    sl = pl.ds(r * LANES, LANES)
                    hbuf[cur][e, sl] = hbuf[cur][e, sl] * ev
                hbuf[cur][e, pl.ds(HID, LANES)] = ev
            pltpu.async_copy(hbuf[cur], oacc.at[didxa.at[b & 3]],
                             ssem[cur], add=True)

        s_wait((NBLK - 1) % 2)
        plsc.subcore_barrier()
        pltpu.sync_copy(oacc.at[pl.ds(sid * RPT, RPT)],
                        acc_h.at[pl.ds(cid * NPAD + sid * RPT, RPT)])

    acc2 = sc_b2(h2a, adst2t, src, dst, z80)

    outf, _, _ = pl.pallas_call(
        _tc3_body,
        grid=(NB,),
        in_specs=[
            pl.BlockSpec((2, BN, 80), lambda i: (0, i, 0)),
            pl.BlockSpec((BN, 1), lambda i: (i, 0)),
            pl.BlockSpec((1, HID), lambda i: (0, 0)),
            pl.BlockSpec((HID, blin.shape[0]), lambda i: (0, 0)),
            pl.BlockSpec((1, blin.shape[0]), lambda i: (0, 0)),
        ],
        out_specs=[
            pl.BlockSpec((NG, blin.shape[0]), lambda i: (0, 0)),
            pl.BlockSpec((NG, HID), lambda i: (0, 0)),
            pl.BlockSpec((NG, HID), lambda i: (0, 0)),
        ],
        out_shape=[
            jax.ShapeDtypeStruct((NG, blin.shape[0]), f32),
            jax.ShapeDtypeStruct((NG, HID), f32),
            jax.ShapeDtypeStruct((NG, HID), f32),
        ],
    )(acc2.reshape(2, NPAD, 80), batch_p, b2r, Wlin, blr)

    return {"hc": hc, "asrc16": asrc16, "adst16": adst16, "ex": ex,
            "den2": den2, "outc": outc, "h2a": h2a, "adst2t": adst2t,
            "acc2": acc2, "outf": outf}


def kernel(x, edge_index, batch, W1, att_src1, att_dst1, b1,
           W2, att_src2, att_dst2, b2, Wlin, blin):
    return _pipeline(x, edge_index, batch, W1, att_src1, att_dst1, b1,
                     W2, att_src2, att_dst2, b2, Wlin, blin)["outf"]


# speedup vs baseline: 45.9411x; 1.1237x over previous
import functools

import jax
import jax.numpy as jnp
from jax import lax
from jax.experimental import pallas as pl
from jax.experimental.pallas import tpu as pltpu
from jax.experimental.pallas import tpu_sc as plsc

HEADS = 8
HID = 64
NG = 16
LANES = 16
NW = 32
KE = 128


def _tc1_body(x_ref, w_ref, as_ref, ad_ref, h_ref, asrc_ref, adst_ref):
    c = pl.program_id(1)
    h = jnp.dot(x_ref[...], w_ref[...], preferred_element_type=jnp.float32)
    h_ref[0] = h

    @pl.when(c == 0)
    def _():
        asrc_ref[...] = jnp.zeros_like(asrc_ref)
        adst_ref[...] = jnp.zeros_like(adst_ref)

    asrc_ref[...] += jnp.dot(h, as_ref[...], preferred_element_type=jnp.float32)
    adst_ref[...] += jnp.dot(h, ad_ref[...], preferred_element_type=jnp.float32)


def _tc2_body(outc_ref, den_ref, b1_ref, w2_ref, as2_ref, ad2_ref, e2_ref,
              h2a_ref, adst2_ref):
    chunks = []
    for c in range(4):
        raw = outc_ref[c]
        d = den_ref[0, :, 2 * c:2 * c + 2] + den_ref[1, :, 2 * c:2 * c + 2]
        db = jnp.dot(d + 1e-16, e2_ref[...],
                     preferred_element_type=jnp.float32)
        z = raw / db + b1_ref[0, c * 128:(c + 1) * 128]
        chunks.append(jnp.where(z > 0, z, jnp.exp(z) - 1.0))
    h1 = jnp.concatenate(chunks, axis=1)
    h2 = jnp.dot(h1, w2_ref[...], preferred_element_type=jnp.float32)
    a2s = jnp.dot(h2, as2_ref[...], preferred_element_type=jnp.float32)
    a2d = jnp.dot(h2, ad2_ref[...], preferred_element_type=jnp.float32)
    bn = h2.shape[0]
    h2a_ref[...] = jnp.concatenate(
        [h2, jnp.broadcast_to(a2s, (bn, LANES))], axis=1)
    adst2_ref[...] = jnp.broadcast_to(a2d, (bn, LANES))


def _tc3_body(acc_ref, batch_ref, b2_ref, wl_ref, bl_ref,
              out_ref, sums_ref, cnt_ref):
    i = pl.program_id(0)
    rows = acc_ref[0] + acc_ref[1]
    den = rows[:, HID:HID + 1] + 1e-16
    h2o = rows[:, :HID] / den + b2_ref[...]
    bn = h2o.shape[0]
    oh = (batch_ref[...] == lax.broadcasted_iota(jnp.int32, (1, NG), 1)
          ).astype(jnp.float32)

    @pl.when(i == 0)
    def _():
        sums_ref[...] = jnp.zeros_like(sums_ref)
        cnt_ref[...] = jnp.zeros_like(cnt_ref)

    dn = (((0,), (0,)), ((), ()))
    sums_ref[...] += lax.dot_general(oh, h2o, dn,
                                     preferred_element_type=jnp.float32)
    cnt_ref[...] += lax.dot_general(oh, jnp.ones((bn, HID), jnp.float32), dn,
                                    preferred_element_type=jnp.float32)
    pooled = sums_ref[...] / jnp.maximum(cnt_ref[...], 1.0)
    out_ref[...] = (jnp.dot(pooled, wl_ref[...],
                            preferred_element_type=jnp.float32) + bl_ref[...])


def _pipeline(x, edge_index, batch, W1, att_src1, att_dst1, b1,
              W2, att_src2, att_dst2, b2, Wlin, blin):
    f32, i32 = jnp.float32, jnp.int32
    N, F = x.shape
    E0 = edge_index.shape[1]
    ei = edge_index.astype(i32)

    NPAD = ((N + 255) // 256) * 256
    RPT = NPAD // 16
    E1 = E0 + N
    EPT = ((E1 + NW * KE - 1) // (NW * KE)) * KE
    NBLK = EPT // KE
    EPAD = EPT * NW

    loops = jnp.arange(N, dtype=i32)
    src = jnp.concatenate([ei[0], loops,
                           jnp.zeros((EPAD - E1,), i32)])
    dst = jnp.concatenate([ei[1], loops,
                           jnp.full((EPAD - E1,), N, i32)])
    xp = jnp.pad(x, ((0, NPAD - N), (0, 0)))
    batch_p = jnp.pad(batch.astype(i32), (0, NPAD - N),
                      constant_values=NG).reshape(NPAD, 1)

    eyeH = jnp.eye(HEADS, dtype=f32)
    A2s = (att_src1[0][:, :, None] * eyeH[:, None, :]).reshape(HEADS * HID, HEADS)
    A2d = (att_dst1[0][:, :, None] * eyeH[:, None, :]).reshape(HEADS * HID, HEADS)
    A2s = jnp.concatenate([A2s, A2s], axis=1)
    A2d = jnp.concatenate([A2d, A2d], axis=1)
    E2 = jnp.repeat(jnp.eye(2, dtype=f32), 128 // 2, axis=1)
    as2 = att_src2[0, 0].reshape(HID, 1)
    ad2 = att_dst2[0, 0].reshape(HID, 1)
    b1r = b1.reshape(1, HEADS * HID)
    b2r = b2.reshape(1, HID)
    blr = blin.reshape(1, -1)
    z16 = jnp.zeros((RPT, LANES), f32)
    z128 = jnp.zeros((RPT, 128), f32)
    z80 = jnp.zeros((RPT, 80), f32)

    BN = 256
    NB = NPAD // BN

    hc, asrc16, adst16 = pl.pallas_call(
        _tc1_body,
        grid=(NB, 4),
        in_specs=[
            pl.BlockSpec((BN, F), lambda i, c: (i, 0)),
            pl.BlockSpec((F, 128), lambda i, c: (0, c)),
            pl.BlockSpec((128, LANES), lambda i, c: (c, 0)),
            pl.BlockSpec((128, LANES), lambda i, c: (c, 0)),
        ],
        out_specs=[
            pl.BlockSpec((1, BN, 128), lambda i, c: (c, i, 0)),
            pl.BlockSpec((BN, LANES), lambda i, c: (i, 0)),
            pl.BlockSpec((BN, LANES), lambda i, c: (i, 0)),
        ],
        out_shape=[
            jax.ShapeDtypeStruct((4, NPAD, 128), f32),
            jax.ShapeDtypeStruct((NPAD, LANES), f32),
            jax.ShapeDtypeStruct((NPAD, LANES), f32),
        ],
    )(xp, W1, A2s, A2d)

    mesh = plsc.VectorSubcoreMesh(core_axis_name="c", subcore_axis_name="s")

    @functools.partial(
        pl.kernel,
        out_type=(jax.ShapeDtypeStruct((EPAD, LANES), f32),
                  jax.ShapeDtypeStruct((2 * NPAD, LANES), f32)),
        mesh=mesh,
        compiler_params=pltpu.CompilerParams(use_tc_tiling_on_sc=False),
        scratch_types=[
            pltpu.VMEM((4, KE), i32), pltpu.VMEM((4, KE), i32),
            [pltpu.VMEM((KE, LANES), f32)] * 2,
            [pltpu.VMEM((KE, LANES), f32)] * 2,
            pltpu.VMEM_SHARED((NPAD, LANES), f32),
            [pltpu.SemaphoreType.DMA] * 2,
            [pltpu.SemaphoreType.DMA] * 2,
            [pltpu.SemaphoreType.DMA] * 2,
            [pltpu.SemaphoreType.DMA] * 2,
            [pltpu.SemaphoreType.DMA] * 2,
            [pltpu.SemaphoreType.DMA] * 2,
        ],
    )
    def sc_a1(asrc_h, adst_h, src_h, dst_h, z16_h, ex_h, den_h,
              sidxa, didxa, asb, adb, dacc, gsem, asem, ssem, wsem, ism, idm):
        cid = lax.axis_index("c")
        sid = lax.axis_index("s")
        wid = cid * 16 + sid
        ebase = wid * EPT

        def s_wait(s):
            pltpu.make_async_copy(adb[s], dacc.at[didxa.at[0]],
                                  ssem[s]).wait()
            pltpu.make_async_copy(adb[s], ex_h.at[pl.ds(0, KE)],
                                  wsem[s]).wait()

        def compute(cur):
            @plsc.parallel_loop(0, KE, 1, unroll=4)
            def edge(e):
                s = asb[cur][e, pl.ds(0, LANES)] + adb[cur][e, pl.ds(0, LANES)]
                adb[cur][e, pl.ds(0, LANES)] = jnp.exp(
                    jnp.maximum(s, 0.2 * s))

        def emit(cur, b):
            pltpu.async_copy(adb[cur], dacc.at[didxa.at[b & 3]],
                             ssem[cur], add=True)
            pltpu.async_copy(adb[cur], ex_h.at[pl.ds(ebase + b * KE, KE)],
                             wsem[cur])

        pltpu.sync_copy(z16_h, dacc.at[pl.ds(sid * RPT, RPT)])
        plsc.subcore_barrier()
        for b0 in range(2):
            pltpu.sync_copy(src_h.at[pl.ds(ebase + b0 * KE, KE)],
                            sidxa.at[b0])
            pltpu.sync_copy(dst_h.at[pl.ds(ebase + b0 * KE, KE)],
                            didxa.at[b0])
        pltpu.async_copy(asrc_h.at[sidxa.at[0]], asb[0], gsem[0])
        pltpu.async_copy(adst_h.at[didxa.at[0]], adb[0], asem[0])

        def blk2(t, _):
            for ph in range(2):
                b = 2 * t + ph
                cur, nxt = ph, 1 - ph

                @pl.when(b >= 1)
                def _():
                    s_wait(nxt)

                @pl.when(b + 2 < NBLK)
                def _():
                    pltpu.async_copy(
                        src_h.at[pl.ds(ebase + (b + 2) * KE, KE)],
                        sidxa.at[(b + 2) & 3], ism[cur])
                    pltpu.async_copy(
                        dst_h.at[pl.ds(ebase + (b + 2) * KE, KE)],
                        didxa.at[(b + 2) & 3], idm[cur])

                @pl.when(b + 1 < NBLK)
                def _():
                    @pl.when(b + 1 >= 2)
                    def _():
                        pltpu.make_async_copy(
                            src_h.at[pl.ds(ebase, KE)],
                            sidxa.at[0], ism[nxt]).wait()
                        pltpu.make_async_copy(
                            dst_h.at[pl.ds(ebase, KE)],
                            didxa.at[0], idm[nxt]).wait()

                    pltpu.async_copy(asrc_h.at[sidxa.at[(b + 1) & 3]],
                                     asb[nxt], gsem[nxt])
                    pltpu.async_copy(adst_h.at[didxa.at[(b + 1) & 3]],
                                     adb[nxt], asem[nxt])

                pltpu.make_async_copy(asrc_h.at[sidxa.at[0]], asb[cur],
                                      gsem[cur]).wait()
                pltpu.make_async_copy(adst_h.at[didxa.at[0]], adb[cur],
                                      asem[cur]).wait()
                compute(cur)
                emit(cur, b)
            return 0

        lax.fori_loop(0, NBLK // 2, blk2, 0)

        if NBLK % 2 == 1:
            b = NBLK - 1
            cur, nxt = 0, 1
            s_wait(nxt)
            pltpu.make_async_copy(asrc_h.at[sidxa.at[0]], asb[cur],
                                  gsem[cur]).wait()
            pltpu.make_async_copy(adst_h.at[didxa.at[0]], adb[cur],
                                  asem[cur]).wait()
            compute(cur)
            emit(cur, b)

        s_wait((NBLK - 1) % 2)
        plsc.subcore_barrier()
        pltpu.sync_copy(dacc.at[pl.ds(sid * RPT, RPT)],
                        den_h.at[pl.ds(cid * NPAD + sid * RPT, RPT)])

    ex, den2 = sc_a1(asrc16, adst16, src, dst, z16)

    EPT16 = EPAD // 16
    NBLK16 = EPT16 // KE

    @functools.partial(
        pl.kernel,
        out_type=jax.ShapeDtypeStruct((4 * NPAD, 128), f32),
        mesh=mesh,
        compiler_params=pltpu.CompilerParams(use_tc_tiling_on_sc=False),
        scratch_types=[
            pltpu.VMEM((4, KE), i32), pltpu.VMEM((4, KE), i32),
            [pltpu.VMEM((KE, 128), f32)] * 2,
            [pltpu.VMEM((KE, LANES), f32)] * 2,
            pltpu.VMEM_SHARED((NPAD, 128), f32),
            [pltpu.SemaphoreType.DMA] * 2,
            [pltpu.SemaphoreType.DMA] * 2,
            [pltpu.SemaphoreType.DMA] * 2,
            [pltpu.SemaphoreType.DMA] * 2,
            [pltpu.SemaphoreType.DMA] * 2,
        ],
    )
    def sc_b1(hcat_h, exf_h, src4_h, dst_h, z128_h, outc_h,
              sidxa, didxa, hbuf, exb, oacc,
              gsem, esem, ssem, ism, idm):
        cid = lax.axis_index("c")
        sid = lax.axis_index("s")

        def s_wait(s):
            pltpu.make_async_copy(hbuf[s], oacc.at[didxa.at[0]],
                                  ssem[s]).wait()

        for chunk in range(2):
            c = cid * 2 + chunk
            cN = c * NPAD
            c2 = c * 2
            ebase = sid * EPT16
            sbase = c * EPAD + ebase
            pltpu.sync_copy(z128_h, oacc.at[pl.ds(sid * RPT, RPT)])
            plsc.subcore_barrier()
            for b0 in range(2):
                pltpu.sync_copy(src4_h.at[pl.ds(sbase + b0 * KE, KE)],
                                sidxa.at[b0])
                pltpu.sync_copy(dst_h.at[pl.ds(ebase + b0 * KE, KE)],
                                didxa.at[b0])
            pltpu.async_copy(hcat_h.at[sidxa.at[0]], hbuf[0], gsem[0])
            pltpu.async_copy(exf_h.at[pl.ds(ebase, KE)], exb[0], esem[0])

            def blk2(t, _):
                for ph in range(2):
                    b = 2 * t + ph
                    cur, nxt = ph, 1 - ph

                    @pl.when(b >= 1)
                    def _():
                        s_wait(nxt)

                    @pl.when(b + 2 < NBLK16)
                    def _():
                        pltpu.async_copy(
                            src4_h.at[pl.ds(sbase + (b + 2) * KE, KE)],
                            sidxa.at[(b + 2) & 3], ism[cur])
                        pltpu.async_copy(
                            dst_h.at[pl.ds(ebase + (b + 2) * KE, KE)],
                            didxa.at[(b + 2) & 3], idm[cur])

                    @pl.when(b + 1 < NBLK16)
                    def _():
                        @pl.when(b + 1 >= 2)
                        def _():
                            pltpu.make_async_copy(
                                src4_h.at[pl.ds(sbase, KE)],
                                sidxa.at[0], ism[nxt]).wait()
                            pltpu.make_async_copy(
                                dst_h.at[pl.ds(ebase, KE)],
                                didxa.at[0], idm[nxt]).wait()

                        pltpu.async_copy(hcat_h.at[sidxa.at[(b + 1) & 3]],
                                         hbuf[nxt], gsem[nxt])
                        pltpu.async_copy(
                            exf_h.at[pl.ds(ebase + (b + 1) * KE, KE)],
                            exb[nxt], esem[nxt])

                    pltpu.make_async_copy(hcat_h.at[sidxa.at[0]], hbuf[cur],
                                          gsem[cur]).wait()
                    pltpu.make_async_copy(exf_h.at[pl.ds(0, KE)], exb[cur],
                                          esem[cur]).wait()

                    @plsc.parallel_loop(0, KE, 1, unroll=4)
                    def edge(e):
                        row = exb[cur][e, pl.ds(0, LANES)]
                        s0 = row.at[jnp.full((LANES,), c2, i32)].get(
                            mode="promise_in_bounds")
                        s1 = row.at[jnp.full((LANES,), c2 + 1, i32)].get(
                            mode="promise_in_bounds")
                        for r in range(8):
                            sv = s0 if r < 4 else s1
                            sl = pl.ds(r * LANES, LANES)
                            hbuf[cur][e, sl] = hbuf[cur][e, sl] * sv
                    pltpu.async_copy(hbuf[cur], oacc.at[didxa.at[b & 3]],
                                     ssem[cur], add=True)
                return 0

            lax.fori_loop(0, NBLK16 // 2, blk2, 0)
            s_wait((NBLK16 - 1) % 2)
            plsc.subcore_barrier()
            pltpu.sync_copy(oacc.at[pl.ds(sid * RPT, RPT)],
                            outc_h.at[pl.ds(cN + sid * RPT, RPT)])

    src4 = jnp.concatenate([src + c * NPAD for c in range(4)])
    outc = sc_b1(hc.reshape(4 * NPAD, 128), ex, src4, dst, z128)

    h2a, adst2t = pl.pallas_call(
        _tc2_body,
        grid=(NB,),
        in_specs=[
            pl.BlockSpec((4, BN, 128), lambda i: (0, i, 0)),
            pl.BlockSpec((2, BN, LANES), lambda i: (0, i, 0)),
            pl.BlockSpec((1, HEADS * HID), lambda i: (0, 0)),
            pl.BlockSpec((HEADS * HID, HID), lambda i: (0, 0)),
            pl.BlockSpec((HID, 1), lambda i: (0, 0)),
            pl.BlockSpec((HID, 1), lambda i: (0, 0)),
            pl.BlockSpec((2, 128), lambda i: (0, 0)),
        ],
        out_specs=[
            pl.BlockSpec((BN, HID + LANES), lambda i: (i, 0)),
            pl.BlockSpec((BN, LANES), lambda i: (i, 0)),
        ],
        out_shape=[
            jax.ShapeDtypeStruct((NPAD, HID + LANES), f32),
            jax.ShapeDtypeStruct((NPAD, LANES), f32),
        ],
    )(outc.reshape(4, NPAD, 128), den2.reshape(2, NPAD, LANES), b1r, W2,
      as2, ad2, E2)

    @functools.partial(
        pl.kernel,
        out_type=jax.ShapeDtypeStruct((2 * NPAD, 80), f32),
        mesh=mesh,
        compiler_params=pltpu.CompilerParams(use_tc_tiling_on_sc=False),
        scratch_types=[
            pltpu.VMEM((4, KE), i32), pltpu.VMEM((4, KE), i32),
            [pltpu.VMEM((KE, 80), f32)] * 2,
            [pltpu.VMEM((KE, LANES), f32)] * 2,
            pltpu.VMEM_SHARED((NPAD, 80), f32),
            [pltpu.SemaphoreType.DMA] * 2,
            [pltpu.SemaphoreType.DMA] * 2,
            [pltpu.SemaphoreType.DMA] * 2,
            [pltpu.SemaphoreType.DMA] * 2,
            [pltpu.SemaphoreType.DMA] * 2,
        ],
    )
    def sc_b2(h2a_h, adst_h, src_h, dst_h, z80_h, acc_h,
              sidxa, didxa, hbuf, abuf, oacc, gsem, asem, ssem, ism, idm):
        cid = lax.axis_index("c")
        sid = lax.axis_index("s")
        wid = cid * 16 + sid

        def s_wait(s):
            pltpu.make_async_copy(hbuf[s], oacc.at[didxa.at[0]],
                                  ssem[s]).wait()

        ebase = wid * EPT
        pltpu.sync_copy(z80_h, oacc.at[pl.ds(sid * RPT, RPT)])
        plsc.subcore_barrier()
        for b0 in range(2):
            pltpu.sync_copy(src_h.at[pl.ds(ebase + b0 * KE, KE)],
                            sidxa.at[b0])
            pltpu.sync_copy(dst_h.at[pl.ds(ebase + b0 * KE, KE)],
                            didxa.at[b0])
        pltpu.async_copy(h2a_h.at[sidxa.at[0]], hbuf[0], gsem[0])
        pltpu.async_copy(adst_h.at[didxa.at[0]], abuf[0], asem[0])

        def blk2(t, _):
            for ph in range(2):
                b = 2 * t + ph
                cur, nxt = ph, 1 - ph

                @pl.when(b >= 1)
                def _():
                    s_wait(nxt)

                @pl.when(b + 2 < NBLK)
                def _():
                    pltpu.async_copy(
                        src_h.at[pl.ds(ebase + (b + 2) * KE, KE)],
                        sidxa.at[(b + 2) & 3], ism[cur])
                    pltpu.async_copy(
                        dst_h.at[pl.ds(ebase + (b + 2) * KE, KE)],
                        didxa.at[(b + 2) & 3], idm[cur])

                @pl.when(b + 1 < NBLK)
                def _():
                    @pl.when(b + 1 >= 2)
                    def _():
                        pltpu.make_async_copy(
                            src_h.at[pl.ds(ebase, KE)],
                            sidxa.at[0], ism[nxt]).wait()
                        pltpu.make_async_copy(
                            dst_h.at[pl.ds(ebase, KE)],
                            didxa.at[0], idm[nxt]).wait()

                    pltpu.async_copy(h2a_h.at[sidxa.at[(b + 1) & 3]],
                                     hbuf[nxt], gsem[nxt])
                    pltpu.async_copy(adst_h.at[didxa.at[(b + 1) & 3]],
                                     abuf[nxt], asem[nxt])

                pltpu.make_async_copy(h2a_h.at[sidxa.at[0]], hbuf[cur],
                                      gsem[cur]).wait()
                pltpu.make_async_copy(adst_h.at[didxa.at[0]], abuf[cur],
                                      asem[cur]).wait()

                @plsc.parallel_loop(0, KE, 1, unroll=4)
                def edge(e):
                    s = (hbuf[cur][e, pl.ds(HID, LANES)]
                         + abuf[cur][e, pl.ds(0, LANES)])
                    ev = jnp.exp(jnp.maximum(s, 0.2 * s))
                    for r in range(4):
                        sl = pl.ds(r * LANES, LANES)
                        hbuf[cur][e, sl] = hbuf[cur][e, sl] * ev
                    hbuf[cur][e, pl.ds(HID, LANES)] = ev
                pltpu.async_copy(hbuf[cur], oacc.at[didxa.at[b & 3]],
                                 ssem[cur], add=True)
            return 0

        lax.fori_loop(0, NBLK // 2, blk2, 0)

        if NBLK % 2 == 1:
            b = NBLK - 1
            cur, nxt = 0, 1
            s_wait(nxt)
            pltpu.make_async_copy(h2a_h.at[sidxa.at[0]], hbuf[cur],
                                  gsem[cur]).wait()
            pltpu.make_async_copy(adst_h.at[didxa.at[0]], abuf[cur],
                                  asem[cur]).wait()

            @plsc.parallel_loop(0, KE, 1, unroll=4)
            def edge(e):
                s = (hbuf[cur][e, pl.ds(HID, LANES)]
                     + abuf[cur][e, pl.ds(0, LANES)])
                ev = jnp.exp(jnp.maximum(s, 0.2 * s))
                for r in range(4):
                    sl = pl.ds(r * LANES, LANES)
                    hbuf[cur][e, sl] = hbuf[cur][e, sl] * ev
                hbuf[cur][e, pl.ds(HID, LANES)] = ev
            pltpu.async_copy(hbuf[cur], oacc.at[didxa.at[b & 3]],
                             ssem[cur], add=True)

        s_wait((NBLK - 1) % 2)
        plsc.subcore_barrier()
        pltpu.sync_copy(oacc.at[pl.ds(sid * RPT, RPT)],
                        acc_h.at[pl.ds(cid * NPAD + sid * RPT, RPT)])

    acc2 = sc_b2(h2a, adst2t, src, dst, z80)

    outf, _, _ = pl.pallas_call(
        _tc3_body,
        grid=(NB,),
        in_specs=[
            pl.BlockSpec((2, BN, 80), lambda i: (0, i, 0)),
            pl.BlockSpec((BN, 1), lambda i: (i, 0)),
            pl.BlockSpec((1, HID), lambda i: (0, 0)),
            pl.BlockSpec((HID, blin.shape[0]), lambda i: (0, 0)),
            pl.BlockSpec((1, blin.shape[0]), lambda i: (0, 0)),
        ],
        out_specs=[
            pl.BlockSpec((NG, blin.shape[0]), lambda i: (0, 0)),
            pl.BlockSpec((NG, HID), lambda i: (0, 0)),
            pl.BlockSpec((NG, HID), lambda i: (0, 0)),
        ],
        out_shape=[
            jax.ShapeDtypeStruct((NG, blin.shape[0]), f32),
            jax.ShapeDtypeStruct((NG, HID), f32),
            jax.ShapeDtypeStruct((NG, HID), f32),
        ],
    )(acc2.reshape(2, NPAD, 80), batch_p, b2r, Wlin, blr)

    return {"hc": hc, "asrc16": asrc16, "adst16": adst16, "ex": ex,
            "den2": den2, "outc": outc, "h2a": h2a, "adst2t": adst2t,
            "acc2": acc2, "outf": outf}


def kernel(x, edge_index, batch, W1, att_src1, att_dst1, b1,
           W2, att_src2, att_dst2, b2, Wlin, blin):
    return _pipeline(x, edge_index, batch, W1, att_src1, att_dst1, b1,
                     W2, att_src2, att_dst2, b2, Wlin, blin)["outf"]
